# Initial kernel scaffold; baseline (speedup 1.0000x reference)
import functools

import jax
import jax.numpy as jnp
from jax import lax
from jax.experimental import pallas as pl
from jax.experimental.pallas import tpu as pltpu
from jax.experimental.pallas import tpu_sc as plsc

N = 10000
E = 320000
D = 128
NC = 2
NS = 16
CH = 128
NCHUNK = 2528
E_PAD = NCHUNK * CH
NPAD = 10112
STRIPE = NPAD // NS

BM = 1000


def _segsum_body(d, per_tile, tbl, srcix, dstix, out, src_v, dst_v, rows_v,
                 acc_sh):
    cid = lax.axis_index("c")
    sid = lax.axis_index("s")
    zero16 = jnp.zeros((16,), jnp.float32)
    for r in range(CH):
        for c in range(d // 16):
            rows_v[r, pl.ds(c * 16, 16)] = zero16
    row0 = sid * STRIPE
    nfull, rem = STRIPE // CH, STRIPE % CH
    for t in range(nfull):
        pltpu.sync_copy(rows_v, acc_sh.at[pl.ds(row0 + t * CH, CH)])
    if rem:
        pltpu.sync_copy(rows_v.at[pl.ds(0, rem)],
                        acc_sh.at[pl.ds(row0 + nfull * CH, rem)])
    cb = sid * per_tile
    pltpu.sync_copy(srcix.at[cid, pl.ds(cb, per_tile)], src_v)
    pltpu.sync_copy(dstix.at[cid, pl.ds(cb, per_tile)], dst_v)
    plsc.subcore_barrier()

    def body(j, carry):
        pltpu.sync_copy(tbl.at[src_v.at[j]], rows_v)
        pltpu.sync_copy(rows_v, acc_sh.at[dst_v.at[j]], add=True)
        return carry

    lax.fori_loop(0, per_tile, body, 0)
    plsc.subcore_barrier()
    pltpu.sync_copy(acc_sh.at[pl.ds(row0, STRIPE)],
                    out.at[cid, pl.ds(row0, STRIPE)])


def _make_segsum(per_core_chunks):
    per_tile = per_core_chunks // NS
    mesh = plsc.VectorSubcoreMesh(core_axis_name="c", subcore_axis_name="s",
                                  num_cores=NC, num_subcores=NS)
    return functools.partial(
        pl.kernel,
        out_type=jax.ShapeDtypeStruct((NC, NPAD, D), jnp.float32),
        mesh=mesh,
        scratch_types=[
            pltpu.VMEM((per_tile, CH), jnp.int32),
            pltpu.VMEM((per_tile, CH), jnp.int32),
            pltpu.VMEM((CH, D), jnp.float32),
            pltpu.VMEM_SHARED((NPAD, D), jnp.float32),
        ],
    )(functools.partial(_segsum_body, D, per_tile))


def _deg_body(per_tile, dstix, out, dst_v, buf_v, acc_sh):
    cid = lax.axis_index("c")
    sid = lax.axis_index("s")
    zero16 = jnp.zeros((16,), jnp.float32)
    for r in range(CH):
        buf_v[r, pl.ds(0, 16)] = zero16
    row0 = sid * STRIPE
    nfull, rem = STRIPE // CH, STRIPE % CH
    for t in range(nfull):
        pltpu.sync_copy(buf_v, acc_sh.at[pl.ds(row0 + t * CH, CH)])
    if rem:
        pltpu.sync_copy(buf_v.at[pl.ds(0, rem)],
                        acc_sh.at[pl.ds(row0 + nfull * CH, rem)])
    one16 = jnp.ones((16,), jnp.float32)
    for r in range(CH):
        buf_v[r, pl.ds(0, 16)] = one16
    pltpu.sync_copy(dstix.at[cid, pl.ds(sid * per_tile, per_tile)], dst_v)
    plsc.subcore_barrier()

    def body(j, carry):
        pltpu.sync_copy(buf_v, acc_sh.at[dst_v.at[j]], add=True)
        return carry

    lax.fori_loop(0, per_tile, body, 0)
    plsc.subcore_barrier()
    pltpu.sync_copy(acc_sh.at[pl.ds(row0, STRIPE)],
                    out.at[cid, pl.ds(row0, STRIPE)])


def _make_deg(per_core_chunks):
    per_tile = per_core_chunks // NS
    mesh = plsc.VectorSubcoreMesh(core_axis_name="c", subcore_axis_name="s",
                                  num_cores=NC, num_subcores=NS)
    return functools.partial(
        pl.kernel,
        out_type=jax.ShapeDtypeStruct((NC, NPAD, 16), jnp.float32),
        mesh=mesh,
        scratch_types=[
            pltpu.VMEM((per_tile, CH), jnp.int32),
            pltpu.VMEM((CH, 16), jnp.float32),
            pltpu.VMEM_SHARED((NPAD, 16), jnp.float32),
        ],
    )(functools.partial(_deg_body, per_tile))


def _dinv(degp0, degp1):
    return lax.rsqrt(degp0[:, 0:1] + degp1[:, 0:1] + 1.0)


_HI = lax.Precision.HIGHEST


def _k0_body(degp_ref, x_ref, w_ref, g_ref):
    dinv = _dinv(degp_ref[0], degp_ref[1])
    g_ref[...] = dinv * jnp.dot(x_ref[...], w_ref[...],
                                preferred_element_type=jnp.float32,
                                precision=_HI)


def _k1_body(p_ref, g_ref, b_ref, degp_ref, w_ref, o_ref):
    dinv = _dinv(degp_ref[0], degp_ref[1])
    x = jnp.maximum(dinv * (p_ref[0] + p_ref[1] + g_ref[...]) + b_ref[...],
                    0.0)
    o_ref[...] = dinv * jnp.dot(x, w_ref[...],
                                preferred_element_type=jnp.float32,
                                precision=_HI)


def _k2_body(p_ref, g_ref, b_ref, degp_ref, w_ref, o_ref):
    dinv = _dinv(degp_ref[0], degp_ref[1])
    x = jnp.maximum(dinv * (p_ref[0] + p_ref[1] + g_ref[...]) + b_ref[...],
                    0.0)
    o_ref[...] = dinv * jnp.dot(x, w_ref[0],
                                preferred_element_type=jnp.float32,
                                precision=_HI)


def _k3_body(pp_ref, g_ref, b_ref, degp_ref, w_ref, o_ref):
    dinv = _dinv(degp_ref[0], degp_ref[1])
    x = jnp.maximum(dinv * (pp_ref[0] + g_ref[...]) + b_ref[0], 0.0)
    o_ref[...] = dinv * jnp.dot(x, w_ref[0],
                                preferred_element_type=jnp.float32,
                                precision=_HI)


def _k4_body(pa_ref, pb_ref, ga_ref, gb_ref, ba_ref, bb_ref, degp_ref,
             eps_ref, mu_ref, z_ref):
    dinv = _dinv(degp_ref[0], degp_ref[1])
    mu = jnp.maximum(dinv * (pa_ref[0] + ga_ref[...]) + ba_ref[...], 0.0)
    logvar = jnp.maximum(dinv * (pb_ref[0] + gb_ref[...]) + bb_ref[...], 0.0)
    mu_ref[...] = mu
    z_ref[...] = mu + eps_ref[...] * jnp.exp(0.5 * logvar)


def _dec_body(zr_ref, zc_ref, o_ref):
    o_ref[...] = lax.dot_general(
        zr_ref[...], zc_ref[...], (((1,), (1,)), ((), ())),
        preferred_element_type=jnp.float32, precision=_HI)


def kernel(X, edge_index, W0, b0, W1, b1, W2, b2, W3, b3, W4, b4, W5, b5, W6,
           b6):
    f32 = jnp.float32
    src = edge_index[0]
    dst = edge_index[1]
    pad = E_PAD - E
    srcp = jnp.concatenate([src, jnp.zeros((pad,), jnp.int32)])
    dstp = jnp.concatenate([dst, jnp.full((pad,), N, jnp.int32)])
    src2 = srcp.reshape(NCHUNK, CH)
    dst2 = dstp.reshape(NCHUNK, CH)
    src_trunk = src2.reshape(NC, NCHUNK // NC, CH)
    dst_trunk = dst2.reshape(NC, NCHUNK // NC, CH)
    src_pair = jnp.stack([src2, src2 + N])
    dst_pair = jnp.stack([dst2, dst2])
    eps = jax.random.normal(jax.random.key(42), (N, D), dtype=f32)

    seg_trunk = _make_segsum(NCHUNK // NC)
    seg_pair = _make_segsum(NCHUNK)
    deg_k = _make_deg(NCHUNK // NC)

    degp = deg_k(dst_trunk)

    grid10 = N // BM
    spec_degp = pl.BlockSpec((NC, BM, 16), lambda i: (0, i, 0))
    spec_rows = pl.BlockSpec((BM, D), lambda i: (i, 0))
    spec_w = pl.BlockSpec((D, D), lambda i: (0, 0))
    spec_b = pl.BlockSpec((1, D), lambda i: (0, 0))
    spec_p = pl.BlockSpec((NC, BM, D), lambda i: (0, i, 0))

    b0r, b1r, b2r = b0.reshape(1, D), b1.reshape(1, D), b2.reshape(1, D)

    g = pl.pallas_call(
        _k0_body, grid=(grid10,),
        in_specs=[spec_degp, spec_rows, spec_w],
        out_specs=spec_rows,
        out_shape=jax.ShapeDtypeStruct((N, D), f32),
    )(degp, X, W0)

    for b_i, w_next in ((b0r, W1), (b1r, W2)):
        p = seg_trunk(g, src_trunk, dst_trunk)
        g = pl.pallas_call(
            _k1_body, grid=(grid10,),
            in_specs=[spec_p, spec_rows, spec_b, spec_degp, spec_w],
            out_specs=spec_rows,
            out_shape=jax.ShapeDtypeStruct((N, D), f32),
        )(p, g, b_i, degp, w_next)

    p2 = seg_trunk(g, src_trunk, dst_trunk)
    w35 = jnp.stack([W3, W5])
    g35 = pl.pallas_call(
        _k2_body, grid=(2 * grid10,),
        in_specs=[
            pl.BlockSpec((NC, BM, D), lambda i: (0, i % grid10, 0)),
            pl.BlockSpec((BM, D), lambda i: (i % grid10, 0)),
            spec_b,
            pl.BlockSpec((NC, BM, 16), lambda i: (0, i % grid10, 0)),
            pl.BlockSpec((1, D, D), lambda i: (i // grid10, 0, 0)),
        ],
        out_specs=pl.BlockSpec((BM, D), lambda i: (i, 0)),
        out_shape=jax.ShapeDtypeStruct((2 * N, D), f32),
    )(p2, g, b2r, degp, w35)

    pp = seg_pair(g35, src_pair, dst_pair)

    b35 = jnp.stack([b3.reshape(1, D), b5.reshape(1, D)])
    w46 = jnp.stack([W4, W6])
    g46 = pl.pallas_call(
        _k3_body, grid=(2 * grid10,),
        in_specs=[
            pl.BlockSpec((1, BM, D), lambda i: (i // grid10, i % grid10, 0)),
            pl.BlockSpec((BM, D), lambda i: (i, 0)),
            pl.BlockSpec((1, 1, D), lambda i: (i // grid10, 0, 0)),
            pl.BlockSpec((NC, BM, 16), lambda i: (0, i % grid10, 0)),
            pl.BlockSpec((1, D, D), lambda i: (i // grid10, 0, 0)),
        ],
        out_specs=pl.BlockSpec((BM, D), lambda i: (i, 0)),
        out_shape=jax.ShapeDtypeStruct((2 * N, D), f32),
    )(pp, g35, b35, degp, w46)

    pp2 = seg_pair(g46, src_pair, dst_pair)

    mu, z = pl.pallas_call(
        _k4_body, grid=(grid10,),
        in_specs=[
            pl.BlockSpec((1, BM, D), lambda i: (0, i, 0)),
            pl.BlockSpec((1, BM, D), lambda i: (1, i, 0)),
            pl.BlockSpec((BM, D), lambda i: (i, 0)),
            pl.BlockSpec((BM, D), lambda i: (grid10 + i, 0)),
            spec_b, spec_b, spec_degp, spec_rows,
        ],
        out_specs=[spec_rows, spec_rows],
        out_shape=[jax.ShapeDtypeStruct((N, D), f32),
                   jax.ShapeDtypeStruct((N, D), f32)],
    )(pp2, pp2, g46, g46, b4.reshape(1, D), b6.reshape(1, D), degp, eps)

    BN = 2000
    adj = pl.pallas_call(
        _dec_body, grid=(N // BM, N // BN),
        in_specs=[
            pl.BlockSpec((BM, D), lambda i, j: (i, 0)),
            pl.BlockSpec((BN, D), lambda i, j: (j, 0)),
        ],
        out_specs=pl.BlockSpec((BM, BN), lambda i, j: (i, j)),
        out_shape=jax.ShapeDtypeStruct((N, N), f32),
    )(z, z)

    return (adj, mu, mu)


# speedup vs baseline: 5.8644x; 5.8644x over previous
import functools

import jax
import jax.numpy as jnp
from jax import lax
from jax.experimental import pallas as pl
from jax.experimental.pallas import tpu as pltpu
from jax.experimental.pallas import tpu_sc as plsc

N = 10000
E = 320000
D = 128
NC = 2
NS = 16
CH = 128
NCHUNK = 2560
E_PAD = NCHUNK * CH
NPAD = 10112
STRIPE = NPAD // NS

BM = 1000


IB = 16


def _segsum_body(d, per_tile, tbl, srcix, dstix, out, src_v, dst_v, rows_v,
                 acc_sh):
    cid = lax.axis_index("c")
    sid = lax.axis_index("s")
    zero16 = jnp.zeros((16,), jnp.float32)
    for r in range(CH):
        for c in range(d // 16):
            rows_v[r, pl.ds(c * 16, 16)] = zero16
    row0 = sid * STRIPE
    nfull, rem = STRIPE // CH, STRIPE % CH
    for t in range(nfull):
        pltpu.sync_copy(rows_v, acc_sh.at[pl.ds(row0 + t * CH, CH)])
    if rem:
        pltpu.sync_copy(rows_v.at[pl.ds(0, rem)],
                        acc_sh.at[pl.ds(row0 + nfull * CH, rem)])
    cb = sid * per_tile
    plsc.subcore_barrier()

    def body(j, carry):
        pltpu.sync_copy(tbl.at[src_v.at[j]], rows_v)
        pltpu.sync_copy(rows_v, acc_sh.at[dst_v.at[j]], add=True)
        return carry

    for t in range(per_tile // IB):
        pltpu.sync_copy(srcix.at[cid, pl.ds(cb + t * IB, IB)], src_v)
        pltpu.sync_copy(dstix.at[cid, pl.ds(cb + t * IB, IB)], dst_v)
        lax.fori_loop(0, IB, body, 0)
    plsc.subcore_barrier()
    pltpu.sync_copy(acc_sh.at[pl.ds(row0, STRIPE)],
                    out.at[cid, pl.ds(row0, STRIPE)])


def _make_segsum(per_core_chunks):
    per_tile = per_core_chunks // NS
    mesh = plsc.VectorSubcoreMesh(core_axis_name="c", subcore_axis_name="s",
                                  num_cores=NC, num_subcores=NS)
    return functools.partial(
        pl.kernel,
        out_type=jax.ShapeDtypeStruct((NC, NPAD, D), jnp.float32),
        mesh=mesh,
        scratch_types=[
            pltpu.VMEM((IB, CH), jnp.int32),
            pltpu.VMEM((IB, CH), jnp.int32),
            pltpu.VMEM((CH, D), jnp.float32),
            pltpu.VMEM_SHARED((NPAD, D), jnp.float32),
        ],
    )(functools.partial(_segsum_body, D, per_tile))


DW = 128


def _deg_body(per_tile, dstix, out, dst_v, buf_v, acc_sh):
    cid = lax.axis_index("c")
    sid = lax.axis_index("s")
    zero16 = jnp.zeros((16,), jnp.float32)
    for r in range(CH):
        for c in range(DW // 16):
            buf_v[r, pl.ds(c * 16, 16)] = zero16
    row0 = sid * STRIPE
    nfull, rem = STRIPE // CH, STRIPE % CH
    for t in range(nfull):
        pltpu.sync_copy(buf_v, acc_sh.at[pl.ds(row0 + t * CH, CH)])
    if rem:
        pltpu.sync_copy(buf_v.at[pl.ds(0, rem)],
                        acc_sh.at[pl.ds(row0 + nfull * CH, rem)])
    one16 = jnp.ones((16,), jnp.float32)
    for r in range(CH):
        buf_v[r, pl.ds(0, 16)] = one16
    pltpu.sync_copy(dstix.at[cid, pl.ds(sid * per_tile, per_tile)], dst_v)
    plsc.subcore_barrier()

    def body(j, carry):
        pltpu.sync_copy(buf_v, acc_sh.at[dst_v.at[j]], add=True)
        return carry

    lax.fori_loop(0, per_tile, body, 0)
    plsc.subcore_barrier()
    pltpu.sync_copy(acc_sh.at[pl.ds(row0, STRIPE)],
                    out.at[cid, pl.ds(row0, STRIPE)])


def _make_deg(per_core_chunks):
    per_tile = per_core_chunks // NS
    mesh = plsc.VectorSubcoreMesh(core_axis_name="c", subcore_axis_name="s",
                                  num_cores=NC, num_subcores=NS)
    return functools.partial(
        pl.kernel,
        out_type=jax.ShapeDtypeStruct((NC, NPAD, DW), jnp.float32),
        mesh=mesh,
        scratch_types=[
            pltpu.VMEM((per_tile, CH), jnp.int32),
            pltpu.VMEM((CH, DW), jnp.float32),
            pltpu.VMEM_SHARED((NPAD, DW), jnp.float32),
        ],
    )(functools.partial(_deg_body, per_tile))


def _dinv(degp0, degp1):
    return lax.rsqrt(degp0[:, 0:1] + degp1[:, 0:1] + 1.0)


_HI = lax.Precision.HIGHEST


def _k0_body(degp_ref, x_ref, w_ref, g_ref):
    dinv = _dinv(degp_ref[0], degp_ref[1])
    g_ref[...] = dinv * jnp.dot(x_ref[...], w_ref[...],
                                preferred_element_type=jnp.float32,
                                precision=_HI)


def _k1_body(p_ref, g_ref, b_ref, degp_ref, w_ref, o_ref):
    dinv = _dinv(degp_ref[0], degp_ref[1])
    x = jnp.maximum(dinv * (p_ref[0] + p_ref[1] + g_ref[...]) + b_ref[...],
                    0.0)
    o_ref[...] = dinv * jnp.dot(x, w_ref[...],
                                preferred_element_type=jnp.float32,
                                precision=_HI)


def _k2_body(p_ref, g_ref, b_ref, degp_ref, w_ref, o_ref):
    dinv = _dinv(degp_ref[0], degp_ref[1])
    x = jnp.maximum(dinv * (p_ref[0] + p_ref[1] + g_ref[...]) + b_ref[...],
                    0.0)
    o_ref[...] = dinv * jnp.dot(x, w_ref[0],
                                preferred_element_type=jnp.float32,
                                precision=_HI)


def _k3_body(pp_ref, g_ref, b_ref, degp_ref, w_ref, o_ref):
    dinv = _dinv(degp_ref[0], degp_ref[1])
    x = jnp.maximum(dinv * (pp_ref[0] + g_ref[...]) + b_ref[0], 0.0)
    o_ref[...] = dinv * jnp.dot(x, w_ref[0],
                                preferred_element_type=jnp.float32,
                                precision=_HI)


def _k4_body(pa_ref, pb_ref, ga_ref, gb_ref, ba_ref, bb_ref, degp_ref,
             eps_ref, mu_ref, z_ref):
    dinv = _dinv(degp_ref[0], degp_ref[1])
    mu = jnp.maximum(dinv * (pa_ref[0] + ga_ref[...]) + ba_ref[...], 0.0)
    logvar = jnp.maximum(dinv * (pb_ref[0] + gb_ref[...]) + bb_ref[...], 0.0)
    mu_ref[...] = mu
    z_ref[...] = mu + eps_ref[...] * jnp.exp(0.5 * logvar)


def _dec_body(zr_ref, zc_ref, o_ref):
    o_ref[...] = lax.dot_general(
        zr_ref[...], zc_ref[...], (((1,), (1,)), ((), ())),
        preferred_element_type=jnp.float32, precision=_HI)


def kernel(X, edge_index, W0, b0, W1, b1, W2, b2, W3, b3, W4, b4, W5, b5, W6,
           b6):
    f32 = jnp.float32
    src = edge_index[0]
    dst = edge_index[1]
    pad = E_PAD - E
    srcp = jnp.concatenate([src, jnp.zeros((pad,), jnp.int32)])
    dstp = jnp.concatenate([dst, jnp.full((pad,), N, jnp.int32)])
    src2 = srcp.reshape(NCHUNK, CH)
    dst2 = dstp.reshape(NCHUNK, CH)
    src_trunk = src2.reshape(NC, NCHUNK // NC, CH)
    dst_trunk = dst2.reshape(NC, NCHUNK // NC, CH)
    src_pair = jnp.stack([src2, src2 + N])
    dst_pair = jnp.stack([dst2, dst2])
    eps = jax.random.normal(jax.random.key(42), (N, D), dtype=f32)

    seg_trunk = _make_segsum(NCHUNK // NC)
    seg_pair = _make_segsum(NCHUNK)
    deg_k = _make_deg(NCHUNK // NC)

    degp = deg_k(dst_trunk)

    grid10 = N // BM
    spec_degp = pl.BlockSpec((NC, BM, DW), lambda i: (0, i, 0))
    spec_rows = pl.BlockSpec((BM, D), lambda i: (i, 0))
    spec_w = pl.BlockSpec((D, D), lambda i: (0, 0))
    spec_b = pl.BlockSpec((1, D), lambda i: (0, 0))
    spec_p = pl.BlockSpec((NC, BM, D), lambda i: (0, i, 0))

    b0r, b1r, b2r = b0.reshape(1, D), b1.reshape(1, D), b2.reshape(1, D)

    g = pl.pallas_call(
        _k0_body, grid=(grid10,),
        in_specs=[spec_degp, spec_rows, spec_w],
        out_specs=spec_rows,
        out_shape=jax.ShapeDtypeStruct((N, D), f32),
    )(degp, X, W0)

    for b_i, w_next in ((b0r, W1), (b1r, W2)):
        p = seg_trunk(g, src_trunk, dst_trunk)
        g = pl.pallas_call(
            _k1_body, grid=(grid10,),
            in_specs=[spec_p, spec_rows, spec_b, spec_degp, spec_w],
            out_specs=spec_rows,
            out_shape=jax.ShapeDtypeStruct((N, D), f32),
        )(p, g, b_i, degp, w_next)

    p2 = seg_trunk(g, src_trunk, dst_trunk)
    w35 = jnp.stack([W3, W5])
    g35 = pl.pallas_call(
        _k2_body, grid=(2 * grid10,),
        in_specs=[
            pl.BlockSpec((NC, BM, D), lambda i: (0, i % grid10, 0)),
            pl.BlockSpec((BM, D), lambda i: (i % grid10, 0)),
            spec_b,
            pl.BlockSpec((NC, BM, DW), lambda i: (0, i % grid10, 0)),
            pl.BlockSpec((1, D, D), lambda i: (i // grid10, 0, 0)),
        ],
        out_specs=pl.BlockSpec((BM, D), lambda i: (i, 0)),
        out_shape=jax.ShapeDtypeStruct((2 * N, D), f32),
    )(p2, g, b2r, degp, w35)

    pp = seg_pair(g35, src_pair, dst_pair)

    b35 = jnp.stack([b3.reshape(1, D), b5.reshape(1, D)])
    w46 = jnp.stack([W4, W6])
    g46 = pl.pallas_call(
        _k3_body, grid=(2 * grid10,),
        in_specs=[
            pl.BlockSpec((1, BM, D), lambda i: (i // grid10, i % grid10, 0)),
            pl.BlockSpec((BM, D), lambda i: (i, 0)),
            pl.BlockSpec((1, 1, D), lambda i: (i // grid10, 0, 0)),
            pl.BlockSpec((NC, BM, DW), lambda i: (0, i % grid10, 0)),
            pl.BlockSpec((1, D, D), lambda i: (i // grid10, 0, 0)),
        ],
        out_specs=pl.BlockSpec((BM, D), lambda i: (i, 0)),
        out_shape=jax.ShapeDtypeStruct((2 * N, D), f32),
    )(pp, g35, b35, degp, w46)

    pp2 = seg_pair(g46, src_pair, dst_pair)

    mu, z = pl.pallas_call(
        _k4_body, grid=(grid10,),
        in_specs=[
            pl.BlockSpec((1, BM, D), lambda i: (0, i, 0)),
            pl.BlockSpec((1, BM, D), lambda i: (1, i, 0)),
            pl.BlockSpec((BM, D), lambda i: (i, 0)),
            pl.BlockSpec((BM, D), lambda i: (grid10 + i, 0)),
            spec_b, spec_b, spec_degp, spec_rows,
        ],
        out_specs=[spec_rows, spec_rows],
        out_shape=[jax.ShapeDtypeStruct((N, D), f32),
                   jax.ShapeDtypeStruct((N, D), f32)],
    )(pp2, pp2, g46, g46, b4.reshape(1, D), b6.reshape(1, D), degp, eps)

    DM, DN = 1024, 2048
    adj = pl.pallas_call(
        _dec_body, grid=(pl.cdiv(N, DM), pl.cdiv(N, DN)),
        in_specs=[
            pl.BlockSpec((DM, D), lambda i, j: (i, 0)),
            pl.BlockSpec((DN, D), lambda i, j: (j, 0)),
        ],
        out_specs=pl.BlockSpec((DM, DN), lambda i, j: (i, j)),
        out_shape=jax.ShapeDtypeStruct((N, N), f32),
    )(z, z)

    return (adj, mu, mu)


# speedup vs baseline: 6.3328x; 1.0799x over previous
import functools

import jax
import jax.numpy as jnp
from jax import lax
from jax.experimental import pallas as pl
from jax.experimental.pallas import tpu as pltpu
from jax.experimental.pallas import tpu_sc as plsc

N = 10000
E = 320000
D = 128
NC = 2
NS = 16
CH = 128
NCHUNK = 2560
E_PAD = NCHUNK * CH
NPAD = 10112
STRIPE = NPAD // NS

BM = 1000


IB = 8


def _segsum_body(d, per_tile, tbl, srcix, dstix, out, src_v, dst_v, rows_v,
                 acc_sh, sem_g, sem_s):
    cid = lax.axis_index("c")
    sid = lax.axis_index("s")
    nbatch = per_tile // IB
    zero16 = jnp.zeros((16,), jnp.float32)
    for r in range(CH):
        for c in range(d // 16):
            rows_v[0, r, pl.ds(c * 16, 16)] = zero16
    row0 = sid * STRIPE
    nfull, rem = STRIPE // CH, STRIPE % CH
    for t in range(nfull):
        pltpu.sync_copy(rows_v.at[0], acc_sh.at[pl.ds(row0 + t * CH, CH)])
    if rem:
        pltpu.sync_copy(rows_v.at[0, pl.ds(0, rem)],
                        acc_sh.at[pl.ds(row0 + nfull * CH, rem)])
    cb = sid * per_tile
    plsc.subcore_barrier()

    def stage(b, parity):
        pltpu.sync_copy(srcix.at[cid, pl.ds(cb + b * IB, IB)],
                        src_v.at[parity])
        pltpu.sync_copy(dstix.at[cid, pl.ds(cb + b * IB, IB)],
                        dst_v.at[parity])

    def gather(parity, j, buf):
        return pltpu.make_async_copy(tbl.at[src_v.at[parity, j]],
                                     rows_v.at[buf], sem_g)

    def scatter(parity, j, buf):
        return pltpu.make_async_copy(rows_v.at[buf],
                                     acc_sh.at[dst_v.at[parity, j]], sem_s)

    stage(0, 0)
    gather(0, 0, 0).start()

    def batch_body(b, carry):
        pb = b % 2
        for j in range(IB):
            buf = j % 2
            k = b * IB + j
            gather(pb, j, buf).wait()
            @pl.when(k > 0)
            def _():
                scatter(pb, j, 1 - buf).wait()
            if j == 0:
                stage((b + 1) % nbatch, 1 - pb)
            nxt_j = (j + 1) % IB
            gather(pb if j + 1 < IB else 1 - pb, nxt_j, 1 - buf).start()
            scatter(pb, j, buf).start(add=True)
        return carry

    lax.fori_loop(0, nbatch, batch_body, 0)
    gather(0, 0, 0).wait()
    scatter(0, IB - 1, (IB - 1) % 2).wait()
    plsc.subcore_barrier()
    pltpu.sync_copy(acc_sh.at[pl.ds(row0, STRIPE)],
                    out.at[cid, pl.ds(row0, STRIPE)])


def _make_segsum(per_core_chunks):
    per_tile = per_core_chunks // NS
    mesh = plsc.VectorSubcoreMesh(core_axis_name="c", subcore_axis_name="s",
                                  num_cores=NC, num_subcores=NS)
    return functools.partial(
        pl.kernel,
        out_type=jax.ShapeDtypeStruct((NC, NPAD, D), jnp.float32),
        mesh=mesh,
        scratch_types=[
            pltpu.VMEM((2, IB, CH), jnp.int32),
            pltpu.VMEM((2, IB, CH), jnp.int32),
            pltpu.VMEM((2, CH, D), jnp.float32),
            pltpu.VMEM_SHARED((NPAD, D), jnp.float32),
            pltpu.SemaphoreType.DMA,
            pltpu.SemaphoreType.DMA,
        ],
    )(functools.partial(_segsum_body, D, per_tile))


DW = 128


def _deg_body(per_tile, dstix, out, dst_v, buf_v, acc_sh, sem_s):
    cid = lax.axis_index("c")
    sid = lax.axis_index("s")
    zero16 = jnp.zeros((16,), jnp.float32)
    for r in range(CH):
        for c in range(DW // 16):
            buf_v[r, pl.ds(c * 16, 16)] = zero16
    row0 = sid * STRIPE
    nfull, rem = STRIPE // CH, STRIPE % CH
    for t in range(nfull):
        pltpu.sync_copy(buf_v, acc_sh.at[pl.ds(row0 + t * CH, CH)])
    if rem:
        pltpu.sync_copy(buf_v.at[pl.ds(0, rem)],
                        acc_sh.at[pl.ds(row0 + nfull * CH, rem)])
    one16 = jnp.ones((16,), jnp.float32)
    for r in range(CH):
        buf_v[r, pl.ds(0, 16)] = one16
    cb = sid * per_tile
    plsc.subcore_barrier()

    def batch_body(b, carry):
        pltpu.sync_copy(dstix.at[cid, pl.ds(cb + b * IB, IB)], dst_v)
        for j in range(IB):
            pltpu.make_async_copy(buf_v, acc_sh.at[dst_v.at[j]],
                                  sem_s).start(add=True)
        for j in range(IB):
            pltpu.make_async_copy(buf_v, acc_sh.at[dst_v.at[j]],
                                  sem_s).wait()
        return carry

    lax.fori_loop(0, per_tile // IB, batch_body, 0)
    plsc.subcore_barrier()
    pltpu.sync_copy(acc_sh.at[pl.ds(row0, STRIPE)],
                    out.at[cid, pl.ds(row0, STRIPE)])


def _make_deg(per_core_chunks):
    per_tile = per_core_chunks // NS
    mesh = plsc.VectorSubcoreMesh(core_axis_name="c", subcore_axis_name="s",
                                  num_cores=NC, num_subcores=NS)
    return functools.partial(
        pl.kernel,
        out_type=jax.ShapeDtypeStruct((NC, NPAD, DW), jnp.float32),
        mesh=mesh,
        scratch_types=[
            pltpu.VMEM((IB, CH), jnp.int32),
            pltpu.VMEM((CH, DW), jnp.float32),
            pltpu.VMEM_SHARED((NPAD, DW), jnp.float32),
            pltpu.SemaphoreType.DMA,
        ],
    )(functools.partial(_deg_body, per_tile))


def _dinv(degp0, degp1):
    return lax.rsqrt(degp0[:, 0:1] + degp1[:, 0:1] + 1.0)


_HI = lax.Precision.HIGHEST


def _k0_body(degp_ref, x_ref, w_ref, g_ref):
    dinv = _dinv(degp_ref[0], degp_ref[1])
    g_ref[...] = dinv * jnp.dot(x_ref[...], w_ref[...],
                                preferred_element_type=jnp.float32,
                                precision=_HI)


def _k1_body(p_ref, g_ref, b_ref, degp_ref, w_ref, o_ref):
    dinv = _dinv(degp_ref[0], degp_ref[1])
    x = jnp.maximum(dinv * (p_ref[0] + p_ref[1] + g_ref[...]) + b_ref[...],
                    0.0)
    o_ref[...] = dinv * jnp.dot(x, w_ref[...],
                                preferred_element_type=jnp.float32,
                                precision=_HI)


def _k2_body(p_ref, g_ref, b_ref, degp_ref, w_ref, o_ref):
    dinv = _dinv(degp_ref[0], degp_ref[1])
    x = jnp.maximum(dinv * (p_ref[0] + p_ref[1] + g_ref[...]) + b_ref[...],
                    0.0)
    o_ref[...] = dinv * jnp.dot(x, w_ref[0],
                                preferred_element_type=jnp.float32,
                                precision=_HI)


def _k3_body(pp_ref, g_ref, b_ref, degp_ref, w_ref, o_ref):
    dinv = _dinv(degp_ref[0], degp_ref[1])
    x = jnp.maximum(dinv * (pp_ref[0] + g_ref[...]) + b_ref[0], 0.0)
    o_ref[...] = dinv * jnp.dot(x, w_ref[0],
                                preferred_element_type=jnp.float32,
                                precision=_HI)


def _k4_body(pa_ref, pb_ref, ga_ref, gb_ref, ba_ref, bb_ref, degp_ref,
             eps_ref, mu_ref, z_ref):
    dinv = _dinv(degp_ref[0], degp_ref[1])
    mu = jnp.maximum(dinv * (pa_ref[0] + ga_ref[...]) + ba_ref[...], 0.0)
    logvar = jnp.maximum(dinv * (pb_ref[0] + gb_ref[...]) + bb_ref[...], 0.0)
    mu_ref[...] = mu
    z_ref[...] = mu + eps_ref[...] * jnp.exp(0.5 * logvar)


def _dec_body(zr_ref, zc_ref, o_ref):
    o_ref[...] = lax.dot_general(
        zr_ref[...], zc_ref[...], (((1,), (1,)), ((), ())),
        preferred_element_type=jnp.float32, precision=_HI)


def kernel(X, edge_index, W0, b0, W1, b1, W2, b2, W3, b3, W4, b4, W5, b5, W6,
           b6):
    f32 = jnp.float32
    src = edge_index[0]
    dst = edge_index[1]
    pad = E_PAD - E
    srcp = jnp.concatenate([src, jnp.zeros((pad,), jnp.int32)])
    dstp = jnp.concatenate([dst, jnp.full((pad,), N, jnp.int32)])
    src2 = srcp.reshape(NCHUNK, CH)
    dst2 = dstp.reshape(NCHUNK, CH)
    src_trunk = src2.reshape(NC, NCHUNK // NC, CH)
    dst_trunk = dst2.reshape(NC, NCHUNK // NC, CH)
    src_pair = jnp.stack([src2, src2 + N])
    dst_pair = jnp.stack([dst2, dst2])
    eps = jax.random.normal(jax.random.key(42), (N, D), dtype=f32)

    seg_trunk = _make_segsum(NCHUNK // NC)
    seg_pair = _make_segsum(NCHUNK)
    deg_k = _make_deg(NCHUNK // NC)

    degp = deg_k(dst_trunk)

    grid10 = N // BM
    spec_degp = pl.BlockSpec((NC, BM, DW), lambda i: (0, i, 0))
    spec_rows = pl.BlockSpec((BM, D), lambda i: (i, 0))
    spec_w = pl.BlockSpec((D, D), lambda i: (0, 0))
    spec_b = pl.BlockSpec((1, D), lambda i: (0, 0))
    spec_p = pl.BlockSpec((NC, BM, D), lambda i: (0, i, 0))

    b0r, b1r, b2r = b0.reshape(1, D), b1.reshape(1, D), b2.reshape(1, D)

    g = pl.pallas_call(
        _k0_body, grid=(grid10,),
        in_specs=[spec_degp, spec_rows, spec_w],
        out_specs=spec_rows,
        out_shape=jax.ShapeDtypeStruct((N, D), f32),
    )(degp, X, W0)

    for b_i, w_next in ((b0r, W1), (b1r, W2)):
        p = seg_trunk(g, src_trunk, dst_trunk)
        g = pl.pallas_call(
            _k1_body, grid=(grid10,),
            in_specs=[spec_p, spec_rows, spec_b, spec_degp, spec_w],
            out_specs=spec_rows,
            out_shape=jax.ShapeDtypeStruct((N, D), f32),
        )(p, g, b_i, degp, w_next)

    p2 = seg_trunk(g, src_trunk, dst_trunk)
    w35 = jnp.stack([W3, W5])
    g35 = pl.pallas_call(
        _k2_body, grid=(2 * grid10,),
        in_specs=[
            pl.BlockSpec((NC, BM, D), lambda i: (0, i % grid10, 0)),
            pl.BlockSpec((BM, D), lambda i: (i % grid10, 0)),
            spec_b,
            pl.BlockSpec((NC, BM, DW), lambda i: (0, i % grid10, 0)),
            pl.BlockSpec((1, D, D), lambda i: (i // grid10, 0, 0)),
        ],
        out_specs=pl.BlockSpec((BM, D), lambda i: (i, 0)),
        out_shape=jax.ShapeDtypeStruct((2 * N, D), f32),
    )(p2, g, b2r, degp, w35)

    pp = seg_pair(g35, src_pair, dst_pair)

    b35 = jnp.stack([b3.reshape(1, D), b5.reshape(1, D)])
    w46 = jnp.stack([W4, W6])
    g46 = pl.pallas_call(
        _k3_body, grid=(2 * grid10,),
        in_specs=[
            pl.BlockSpec((1, BM, D), lambda i: (i // grid10, i % grid10, 0)),
            pl.BlockSpec((BM, D), lambda i: (i, 0)),
            pl.BlockSpec((1, 1, D), lambda i: (i // grid10, 0, 0)),
            pl.BlockSpec((NC, BM, DW), lambda i: (0, i % grid10, 0)),
            pl.BlockSpec((1, D, D), lambda i: (i // grid10, 0, 0)),
        ],
        out_specs=pl.BlockSpec((BM, D), lambda i: (i, 0)),
        out_shape=jax.ShapeDtypeStruct((2 * N, D), f32),
    )(pp, g35, b35, degp, w46)

    pp2 = seg_pair(g46, src_pair, dst_pair)

    mu, z = pl.pallas_call(
        _k4_body, grid=(grid10,),
        in_specs=[
            pl.BlockSpec((1, BM, D), lambda i: (0, i, 0)),
            pl.BlockSpec((1, BM, D), lambda i: (1, i, 0)),
            pl.BlockSpec((BM, D), lambda i: (i, 0)),
            pl.BlockSpec((BM, D), lambda i: (grid10 + i, 0)),
            spec_b, spec_b, spec_degp, spec_rows,
        ],
        out_specs=[spec_rows, spec_rows],
        out_shape=[jax.ShapeDtypeStruct((N, D), f32),
                   jax.ShapeDtypeStruct((N, D), f32)],
    )(pp2, pp2, g46, g46, b4.reshape(1, D), b6.reshape(1, D), degp, eps)

    DM, DN = 1024, 2048
    adj = pl.pallas_call(
        _dec_body, grid=(pl.cdiv(N, DM), pl.cdiv(N, DN)),
        in_specs=[
            pl.BlockSpec((DM, D), lambda i, j: (i, 0)),
            pl.BlockSpec((DN, D), lambda i, j: (j, 0)),
        ],
        out_specs=pl.BlockSpec((DM, DN), lambda i, j: (i, j)),
        out_shape=jax.ShapeDtypeStruct((N, N), f32),
    )(z, z)

    return (adj, mu, mu)


# speedup vs baseline: 6.7335x; 1.0633x over previous
import functools

import jax
import jax.numpy as jnp
from jax import lax
from jax.experimental import pallas as pl
from jax.experimental.pallas import tpu as pltpu
from jax.experimental.pallas import tpu_sc as plsc

N = 10000
E = 320000
D = 128
NC = 2
NS = 16
CH = 128
NCHUNK = 2560
E_PAD = NCHUNK * CH
NPAD = 10112
STRIPE = NPAD // NS

BM = 1000


IB = 8


def _segsum_body(d, per_tile, tbl, srcix, dstix, out, src_v, dst_v, rows_v,
                 acc_sh, sem_g, sem_s):
    cid = lax.axis_index("c")
    sid = lax.axis_index("s")
    nbatch = per_tile // IB
    zero16 = jnp.zeros((16,), jnp.float32)
    for r in range(CH):
        for c in range(d // 16):
            rows_v[0, r, pl.ds(c * 16, 16)] = zero16
    row0 = sid * STRIPE
    nfull, rem = STRIPE // CH, STRIPE % CH
    for t in range(nfull):
        pltpu.sync_copy(rows_v.at[0], acc_sh.at[pl.ds(row0 + t * CH, CH)])
    if rem:
        pltpu.sync_copy(rows_v.at[0, pl.ds(0, rem)],
                        acc_sh.at[pl.ds(row0 + nfull * CH, rem)])
    cb = sid * per_tile
    plsc.subcore_barrier()

    def stage(b, parity):
        pltpu.sync_copy(srcix.at[cid, pl.ds(cb + b * IB, IB)],
                        src_v.at[parity])
        pltpu.sync_copy(dstix.at[cid, pl.ds(cb + b * IB, IB)],
                        dst_v.at[parity])

    def gather(parity, j, buf):
        return pltpu.make_async_copy(tbl.at[src_v.at[parity, j]],
                                     rows_v.at[buf], sem_g)

    def scatter(parity, j, buf):
        return pltpu.make_async_copy(rows_v.at[buf],
                                     acc_sh.at[dst_v.at[parity, j]], sem_s)

    stage(0, 0)
    gather(0, 0, 0).start()

    def batch_body(b, carry):
        pb = b % 2
        for j in range(IB):
            buf = j % 2
            k = b * IB + j
            gather(pb, j, buf).wait()
            @pl.when(k > 0)
            def _():
                scatter(pb, j, 1 - buf).wait()
            if j == 0:
                stage((b + 1) % nbatch, 1 - pb)
            nxt_j = (j + 1) % IB
            gather(pb if j + 1 < IB else 1 - pb, nxt_j, 1 - buf).start()
            scatter(pb, j, buf).start(add=True)
        return carry

    lax.fori_loop(0, nbatch, batch_body, 0)
    gather(0, 0, 0).wait()
    scatter(0, IB - 1, (IB - 1) % 2).wait()
    plsc.subcore_barrier()
    pltpu.sync_copy(acc_sh.at[pl.ds(row0, STRIPE)],
                    out.at[cid, pl.ds(row0, STRIPE)])


def _make_segsum(per_core_chunks):
    per_tile = per_core_chunks // NS
    mesh = plsc.VectorSubcoreMesh(core_axis_name="c", subcore_axis_name="s",
                                  num_cores=NC, num_subcores=NS)
    return functools.partial(
        pl.kernel,
        out_type=jax.ShapeDtypeStruct((NC, NPAD, D), jnp.float32),
        mesh=mesh,
        scratch_types=[
            pltpu.VMEM((2, IB, CH), jnp.int32),
            pltpu.VMEM((2, IB, CH), jnp.int32),
            pltpu.VMEM((2, CH, D), jnp.float32),
            pltpu.VMEM_SHARED((NPAD, D), jnp.float32),
            pltpu.SemaphoreType.DMA,
            pltpu.SemaphoreType.DMA,
        ],
    )(functools.partial(_segsum_body, D, per_tile))


DW = 128


def _deg_body(per_tile, dstix, out, dst_v, buf_v, acc_sh, sem_s):
    cid = lax.axis_index("c")
    sid = lax.axis_index("s")
    zero16 = jnp.zeros((16,), jnp.float32)
    for r in range(CH):
        for c in range(DW // 16):
            buf_v[r, pl.ds(c * 16, 16)] = zero16
    row0 = sid * STRIPE
    nfull, rem = STRIPE // CH, STRIPE % CH
    for t in range(nfull):
        pltpu.sync_copy(buf_v, acc_sh.at[pl.ds(row0 + t * CH, CH)])
    if rem:
        pltpu.sync_copy(buf_v.at[pl.ds(0, rem)],
                        acc_sh.at[pl.ds(row0 + nfull * CH, rem)])
    one16 = jnp.ones((16,), jnp.float32)
    for r in range(CH):
        buf_v[r, pl.ds(0, 16)] = one16
    cb = sid * per_tile
    plsc.subcore_barrier()

    def batch_body(b, carry):
        pltpu.sync_copy(dstix.at[cid, pl.ds(cb + b * IB, IB)], dst_v)
        for j in range(IB):
            pltpu.make_async_copy(buf_v, acc_sh.at[dst_v.at[j]],
                                  sem_s).start(add=True)
        for j in range(IB):
            pltpu.make_async_copy(buf_v, acc_sh.at[dst_v.at[j]],
                                  sem_s).wait()
        return carry

    lax.fori_loop(0, per_tile // IB, batch_body, 0)
    plsc.subcore_barrier()
    pltpu.sync_copy(acc_sh.at[pl.ds(row0, STRIPE)],
                    out.at[cid, pl.ds(row0, STRIPE)])


def _make_deg(per_core_chunks):
    per_tile = per_core_chunks // NS
    mesh = plsc.VectorSubcoreMesh(core_axis_name="c", subcore_axis_name="s",
                                  num_cores=NC, num_subcores=NS)
    return functools.partial(
        pl.kernel,
        out_type=jax.ShapeDtypeStruct((NC, NPAD, DW), jnp.float32),
        mesh=mesh,
        scratch_types=[
            pltpu.VMEM((IB, CH), jnp.int32),
            pltpu.VMEM((CH, DW), jnp.float32),
            pltpu.VMEM_SHARED((NPAD, DW), jnp.float32),
            pltpu.SemaphoreType.DMA,
        ],
    )(functools.partial(_deg_body, per_tile))


def _dinv(degp0, degp1):
    return lax.rsqrt(degp0[:, 0:1] + degp1[:, 0:1] + 1.0)


_HI = lax.Precision.HIGHEST


def _k0_body(degp_ref, x_ref, w_ref, g_ref):
    dinv = _dinv(degp_ref[0], degp_ref[1])
    g_ref[...] = dinv * jnp.dot(x_ref[...], w_ref[...],
                                preferred_element_type=jnp.float32,
                                precision=_HI)


def _k1_body(p_ref, g_ref, b_ref, degp_ref, w_ref, o_ref):
    dinv = _dinv(degp_ref[0], degp_ref[1])
    x = jnp.maximum(dinv * (p_ref[0] + p_ref[1] + g_ref[...]) + b_ref[...],
                    0.0)
    o_ref[...] = dinv * jnp.dot(x, w_ref[...],
                                preferred_element_type=jnp.float32,
                                precision=_HI)


def _k2_body(p_ref, g_ref, b_ref, degp_ref, w_ref, o_ref):
    dinv = _dinv(degp_ref[0], degp_ref[1])
    x = jnp.maximum(dinv * (p_ref[0] + p_ref[1] + g_ref[...]) + b_ref[...],
                    0.0)
    o_ref[...] = dinv * jnp.dot(x, w_ref[0],
                                preferred_element_type=jnp.float32,
                                precision=_HI)


def _k3_body(pp_ref, g_ref, b_ref, degp_ref, w_ref, o_ref):
    dinv = _dinv(degp_ref[0], degp_ref[1])
    x = jnp.maximum(dinv * (pp_ref[0] + g_ref[...]) + b_ref[0], 0.0)
    o_ref[...] = dinv * jnp.dot(x, w_ref[0],
                                preferred_element_type=jnp.float32,
                                precision=_HI)


def _k4_body(pa_ref, pb_ref, ga_ref, gb_ref, ba_ref, bb_ref, degp_ref,
             eps_ref, mu_ref, z_ref):
    dinv = _dinv(degp_ref[0], degp_ref[1])
    mu = jnp.maximum(dinv * (pa_ref[0] + ga_ref[...]) + ba_ref[...], 0.0)
    logvar = jnp.maximum(dinv * (pb_ref[0] + gb_ref[...]) + bb_ref[...], 0.0)
    mu_ref[...] = mu
    z_ref[...] = mu + eps_ref[...] * jnp.exp(0.5 * logvar)


def _dec_body(zr_ref, zc_ref, o_ref):
    o_ref[...] = lax.dot_general(
        zr_ref[...], zc_ref[...], (((1,), (1,)), ((), ())),
        preferred_element_type=jnp.float32, precision=_HI)


def kernel(X, edge_index, W0, b0, W1, b1, W2, b2, W3, b3, W4, b4, W5, b5, W6,
           b6):
    f32 = jnp.float32
    src = edge_index[0]
    dst = edge_index[1]
    pad = E_PAD - E
    srcp = jnp.concatenate([src, jnp.zeros((pad,), jnp.int32)])
    dst_fill = N + jnp.arange(pad, dtype=jnp.int32) % (NPAD - N)
    dstp = jnp.concatenate([dst, dst_fill])
    src2 = srcp.reshape(NCHUNK, CH)
    dst2 = dstp.reshape(NCHUNK, CH)
    src_trunk = src2.reshape(NC, NCHUNK // NC, CH)
    dst_trunk = dst2.reshape(NC, NCHUNK // NC, CH)
    src_pair = jnp.stack([src2, src2 + N])
    dst_pair = jnp.stack([dst2, dst2])
    eps = jax.random.normal(jax.random.key(42), (N, D), dtype=f32)

    seg_trunk = _make_segsum(NCHUNK // NC)
    seg_pair = _make_segsum(NCHUNK)
    deg_k = _make_deg(NCHUNK // NC)

    degp = deg_k(dst_trunk)

    grid10 = N // BM
    spec_degp = pl.BlockSpec((NC, BM, DW), lambda i: (0, i, 0))
    spec_rows = pl.BlockSpec((BM, D), lambda i: (i, 0))
    spec_w = pl.BlockSpec((D, D), lambda i: (0, 0))
    spec_b = pl.BlockSpec((1, D), lambda i: (0, 0))
    spec_p = pl.BlockSpec((NC, BM, D), lambda i: (0, i, 0))

    b0r, b1r, b2r = b0.reshape(1, D), b1.reshape(1, D), b2.reshape(1, D)

    g = pl.pallas_call(
        _k0_body, grid=(grid10,),
        in_specs=[spec_degp, spec_rows, spec_w],
        out_specs=spec_rows,
        out_shape=jax.ShapeDtypeStruct((N, D), f32),
    )(degp, X, W0)

    for b_i, w_next in ((b0r, W1), (b1r, W2)):
        p = seg_trunk(g, src_trunk, dst_trunk)
        g = pl.pallas_call(
            _k1_body, grid=(grid10,),
            in_specs=[spec_p, spec_rows, spec_b, spec_degp, spec_w],
            out_specs=spec_rows,
            out_shape=jax.ShapeDtypeStruct((N, D), f32),
        )(p, g, b_i, degp, w_next)

    p2 = seg_trunk(g, src_trunk, dst_trunk)
    w35 = jnp.stack([W3, W5])
    g35 = pl.pallas_call(
        _k2_body, grid=(2 * grid10,),
        in_specs=[
            pl.BlockSpec((NC, BM, D), lambda i: (0, i % grid10, 0)),
            pl.BlockSpec((BM, D), lambda i: (i % grid10, 0)),
            spec_b,
            pl.BlockSpec((NC, BM, DW), lambda i: (0, i % grid10, 0)),
            pl.BlockSpec((1, D, D), lambda i: (i // grid10, 0, 0)),
        ],
        out_specs=pl.BlockSpec((BM, D), lambda i: (i, 0)),
        out_shape=jax.ShapeDtypeStruct((2 * N, D), f32),
    )(p2, g, b2r, degp, w35)

    pp = seg_pair(g35, src_pair, dst_pair)

    b35 = jnp.stack([b3.reshape(1, D), b5.reshape(1, D)])
    w46 = jnp.stack([W4, W6])
    g46 = pl.pallas_call(
        _k3_body, grid=(2 * grid10,),
        in_specs=[
            pl.BlockSpec((1, BM, D), lambda i: (i // grid10, i % grid10, 0)),
            pl.BlockSpec((BM, D), lambda i: (i, 0)),
            pl.BlockSpec((1, 1, D), lambda i: (i // grid10, 0, 0)),
            pl.BlockSpec((NC, BM, DW), lambda i: (0, i % grid10, 0)),
            pl.BlockSpec((1, D, D), lambda i: (i // grid10, 0, 0)),
        ],
        out_specs=pl.BlockSpec((BM, D), lambda i: (i, 0)),
        out_shape=jax.ShapeDtypeStruct((2 * N, D), f32),
    )(pp, g35, b35, degp, w46)

    pp2 = seg_pair(g46, src_pair, dst_pair)

    mu, z = pl.pallas_call(
        _k4_body, grid=(grid10,),
        in_specs=[
            pl.BlockSpec((1, BM, D), lambda i: (0, i, 0)),
            pl.BlockSpec((1, BM, D), lambda i: (1, i, 0)),
            pl.BlockSpec((BM, D), lambda i: (i, 0)),
            pl.BlockSpec((BM, D), lambda i: (grid10 + i, 0)),
            spec_b, spec_b, spec_degp, spec_rows,
        ],
        out_specs=[spec_rows, spec_rows],
        out_shape=[jax.ShapeDtypeStruct((N, D), f32),
                   jax.ShapeDtypeStruct((N, D), f32)],
    )(pp2, pp2, g46, g46, b4.reshape(1, D), b6.reshape(1, D), degp, eps)

    DM, DN = 1024, 2048
    adj = pl.pallas_call(
        _dec_body, grid=(pl.cdiv(N, DM), pl.cdiv(N, DN)),
        in_specs=[
            pl.BlockSpec((DM, D), lambda i, j: (i, 0)),
            pl.BlockSpec((DN, D), lambda i, j: (j, 0)),
        ],
        out_specs=pl.BlockSpec((DM, DN), lambda i, j: (i, j)),
        out_shape=jax.ShapeDtypeStruct((N, N), f32),
    )(z, z)

    return (adj, mu, mu)


# speedup vs baseline: 14.0412x; 2.0853x over previous
import functools

import jax
import jax.numpy as jnp
from jax import lax
from jax.experimental import pallas as pl
from jax.experimental.pallas import tpu as pltpu
from jax.experimental.pallas import tpu_sc as plsc

N = 10000
E = 320000
D = 128
NC = 2
NS = 16
CH = 128
NCHUNK = 2560
E_PAD = NCHUNK * CH
NPAD = 10112
STRIPE = NPAD // NS

BM = 1000


IB = 8


def _segsum_body(d, per_tile, tbl, srcix, dstix, out, src_v, dst_v, rows_v,
                 acc_sh, sem_g, sem_s):
    cid = lax.axis_index("c")
    sid = lax.axis_index("s")
    nbatch = per_tile // IB
    zero16 = jnp.zeros((16,), jnp.float32)
    for r in range(CH):
        for c in range(d // 16):
            rows_v[0, r, pl.ds(c * 16, 16)] = zero16
    row0 = sid * STRIPE
    nfull, rem = STRIPE // CH, STRIPE % CH
    for t in range(nfull):
        pltpu.sync_copy(rows_v.at[0], acc_sh.at[pl.ds(row0 + t * CH, CH)])
    if rem:
        pltpu.sync_copy(rows_v.at[0, pl.ds(0, rem)],
                        acc_sh.at[pl.ds(row0 + nfull * CH, rem)])
    cb = sid * per_tile
    plsc.subcore_barrier()

    def stage(b, parity):
        pltpu.sync_copy(srcix.at[cid, pl.ds(cb + b * IB, IB)],
                        src_v.at[parity])
        pltpu.sync_copy(dstix.at[cid, pl.ds(cb + b * IB, IB)],
                        dst_v.at[parity])

    def gather(parity, j, buf):
        return pltpu.make_async_copy(tbl.at[src_v.at[parity, j]],
                                     rows_v.at[buf], sem_g)

    def scatter(parity, j, buf):
        return pltpu.make_async_copy(rows_v.at[buf],
                                     acc_sh.at[dst_v.at[parity, j]], sem_s)

    stage(0, 0)
    gather(0, 0, 0).start()

    def batch_body(b, carry):
        pb = b % 2
        for j in range(IB):
            buf = j % 2
            k = b * IB + j
            gather(pb, j, buf).wait()
            @pl.when(k > 0)
            def _():
                scatter(pb, j, 1 - buf).wait()
            if j == 0:
                stage((b + 1) % nbatch, 1 - pb)
            nxt_j = (j + 1) % IB
            gather(pb if j + 1 < IB else 1 - pb, nxt_j, 1 - buf).start()
            scatter(pb, j, buf).start(add=True)
        return carry

    lax.fori_loop(0, nbatch, batch_body, 0)
    gather(0, 0, 0).wait()
    scatter(0, IB - 1, (IB - 1) % 2).wait()
    plsc.subcore_barrier()
    pltpu.sync_copy(acc_sh.at[pl.ds(row0, STRIPE)],
                    out.at[cid, pl.ds(row0, STRIPE)])


def _make_segsum(per_core_chunks):
    per_tile = per_core_chunks // NS
    mesh = plsc.VectorSubcoreMesh(core_axis_name="c", subcore_axis_name="s",
                                  num_cores=NC, num_subcores=NS)
    return functools.partial(
        pl.kernel,
        out_type=jax.ShapeDtypeStruct((NC, NPAD, D), jnp.float32),
        mesh=mesh,
        scratch_types=[
            pltpu.VMEM((2, IB, CH), jnp.int32),
            pltpu.VMEM((2, IB, CH), jnp.int32),
            pltpu.VMEM((2, CH, D), jnp.float32),
            pltpu.VMEM_SHARED((NPAD, D), jnp.float32),
            pltpu.SemaphoreType.DMA,
            pltpu.SemaphoreType.DMA,
        ],
    )(functools.partial(_segsum_body, D, per_tile))


DW = 128


def _deg_body(per_tile, dstix, out, dst_v, buf_v, acc_sh, sem_s):
    cid = lax.axis_index("c")
    sid = lax.axis_index("s")
    zero16 = jnp.zeros((16,), jnp.float32)
    for r in range(CH):
        for c in range(DW // 16):
            buf_v[r, pl.ds(c * 16, 16)] = zero16
    row0 = sid * STRIPE
    nfull, rem = STRIPE // CH, STRIPE % CH
    for t in range(nfull):
        pltpu.sync_copy(buf_v, acc_sh.at[pl.ds(row0 + t * CH, CH)])
    if rem:
        pltpu.sync_copy(buf_v.at[pl.ds(0, rem)],
                        acc_sh.at[pl.ds(row0 + nfull * CH, rem)])
    one16 = jnp.ones((16,), jnp.float32)
    for r in range(CH):
        buf_v[r, pl.ds(0, 16)] = one16
    cb = sid * per_tile
    plsc.subcore_barrier()

    def batch_body(b, carry):
        pltpu.sync_copy(dstix.at[cid, pl.ds(cb + b * IB, IB)], dst_v)
        for j in range(IB):
            pltpu.make_async_copy(buf_v, acc_sh.at[dst_v.at[j]],
                                  sem_s).start(add=True)
        for j in range(IB):
            pltpu.make_async_copy(buf_v, acc_sh.at[dst_v.at[j]],
                                  sem_s).wait()
        return carry

    lax.fori_loop(0, per_tile // IB, batch_body, 0)
    plsc.subcore_barrier()
    pltpu.sync_copy(acc_sh.at[pl.ds(row0, STRIPE)],
                    out.at[cid, pl.ds(row0, STRIPE)])


def _make_deg(per_core_chunks):
    per_tile = per_core_chunks // NS
    mesh = plsc.VectorSubcoreMesh(core_axis_name="c", subcore_axis_name="s",
                                  num_cores=NC, num_subcores=NS)
    return functools.partial(
        pl.kernel,
        out_type=jax.ShapeDtypeStruct((NC, NPAD, DW), jnp.float32),
        mesh=mesh,
        scratch_types=[
            pltpu.VMEM((IB, CH), jnp.int32),
            pltpu.VMEM((CH, DW), jnp.float32),
            pltpu.VMEM_SHARED((NPAD, DW), jnp.float32),
            pltpu.SemaphoreType.DMA,
        ],
    )(functools.partial(_deg_body, per_tile))


def _dinv(degp0, degp1):
    return lax.rsqrt(degp0[:, 0:1] + degp1[:, 0:1] + 1.0)


_HI = lax.Precision.HIGHEST


def _k0_body(degp_ref, x_ref, w_ref, g_ref):
    dinv = _dinv(degp_ref[0], degp_ref[1])
    g_ref[...] = dinv * jnp.dot(x_ref[...], w_ref[...],
                                preferred_element_type=jnp.float32,
                                precision=_HI)


def _k1_body(p_ref, g_ref, b_ref, degp_ref, w_ref, o_ref):
    dinv = _dinv(degp_ref[0], degp_ref[1])
    x = jnp.maximum(dinv * (p_ref[0] + p_ref[1] + g_ref[...]) + b_ref[...],
                    0.0)
    o_ref[...] = dinv * jnp.dot(x, w_ref[...],
                                preferred_element_type=jnp.float32,
                                precision=_HI)


def _k2_body(p_ref, g_ref, b_ref, degp_ref, w_ref, o_ref):
    dinv = _dinv(degp_ref[0], degp_ref[1])
    x = jnp.maximum(dinv * (p_ref[0] + p_ref[1] + g_ref[...]) + b_ref[...],
                    0.0)
    o_ref[...] = dinv * jnp.dot(x, w_ref[0],
                                preferred_element_type=jnp.float32,
                                precision=_HI)


def _k3_body(pp_ref, g_ref, b_ref, degp_ref, w_ref, o_ref):
    dinv = _dinv(degp_ref[0], degp_ref[1])
    x = jnp.maximum(dinv * (pp_ref[0] + g_ref[...]) + b_ref[0], 0.0)
    o_ref[...] = dinv * jnp.dot(x, w_ref[0],
                                preferred_element_type=jnp.float32,
                                precision=_HI)


def _k4_body(pa_ref, pb_ref, ga_ref, gb_ref, ba_ref, bb_ref, degp_ref,
             eps_ref, mu_ref, z_ref):
    dinv = _dinv(degp_ref[0], degp_ref[1])
    mu = jnp.maximum(dinv * (pa_ref[0] + ga_ref[...]) + ba_ref[...], 0.0)
    logvar = jnp.maximum(dinv * (pb_ref[0] + gb_ref[...]) + bb_ref[...], 0.0)
    mu_ref[...] = mu
    z_ref[...] = mu + eps_ref[...] * jnp.exp(0.5 * logvar)


def _dec_body(zr_ref, zc_ref, o_ref):
    o_ref[...] = lax.dot_general(
        zr_ref[...], zc_ref[...], (((1,), (1,)), ((), ())),
        preferred_element_type=jnp.float32, precision=_HI)


def kernel(X, edge_index, W0, b0, W1, b1, W2, b2, W3, b3, W4, b4, W5, b5, W6,
           b6):
    f32 = jnp.float32
    src = edge_index[0]
    dst = edge_index[1]
    pad = E_PAD - E
    fill = jnp.arange(pad, dtype=jnp.int32)
    srcp = jnp.concatenate([src, (fill * 131) % N])
    dstp = jnp.concatenate([dst, N + fill % (NPAD - N)])
    src2 = srcp.reshape(NCHUNK, CH)
    dst2 = dstp.reshape(NCHUNK, CH)
    src_trunk = src2.reshape(NC, NCHUNK // NC, CH)
    dst_trunk = dst2.reshape(NC, NCHUNK // NC, CH)
    src_pair = jnp.stack([src2, src2 + N])
    dst_pair = jnp.stack([dst2, dst2])
    eps = jax.random.normal(jax.random.key(42), (N, D), dtype=f32)

    seg_trunk = _make_segsum(NCHUNK // NC)
    seg_pair = _make_segsum(NCHUNK)
    deg_k = _make_deg(NCHUNK // NC)

    degp = deg_k(dst_trunk)

    grid10 = N // BM
    spec_degp = pl.BlockSpec((NC, BM, DW), lambda i: (0, i, 0))
    spec_rows = pl.BlockSpec((BM, D), lambda i: (i, 0))
    spec_w = pl.BlockSpec((D, D), lambda i: (0, 0))
    spec_b = pl.BlockSpec((1, D), lambda i: (0, 0))
    spec_p = pl.BlockSpec((NC, BM, D), lambda i: (0, i, 0))

    b0r, b1r, b2r = b0.reshape(1, D), b1.reshape(1, D), b2.reshape(1, D)

    g = pl.pallas_call(
        _k0_body, grid=(grid10,),
        in_specs=[spec_degp, spec_rows, spec_w],
        out_specs=spec_rows,
        out_shape=jax.ShapeDtypeStruct((N, D), f32),
    )(degp, X, W0)

    for b_i, w_next in ((b0r, W1), (b1r, W2)):
        p = seg_trunk(g, src_trunk, dst_trunk)
        g = pl.pallas_call(
            _k1_body, grid=(grid10,),
            in_specs=[spec_p, spec_rows, spec_b, spec_degp, spec_w],
            out_specs=spec_rows,
            out_shape=jax.ShapeDtypeStruct((N, D), f32),
        )(p, g, b_i, degp, w_next)

    p2 = seg_trunk(g, src_trunk, dst_trunk)
    w35 = jnp.stack([W3, W5])
    g35 = pl.pallas_call(
        _k2_body, grid=(2 * grid10,),
        in_specs=[
            pl.BlockSpec((NC, BM, D), lambda i: (0, i % grid10, 0)),
            pl.BlockSpec((BM, D), lambda i: (i % grid10, 0)),
            spec_b,
            pl.BlockSpec((NC, BM, DW), lambda i: (0, i % grid10, 0)),
            pl.BlockSpec((1, D, D), lambda i: (i // grid10, 0, 0)),
        ],
        out_specs=pl.BlockSpec((BM, D), lambda i: (i, 0)),
        out_shape=jax.ShapeDtypeStruct((2 * N, D), f32),
    )(p2, g, b2r, degp, w35)

    pp = seg_pair(g35, src_pair, dst_pair)

    b35 = jnp.stack([b3.reshape(1, D), b5.reshape(1, D)])
    w46 = jnp.stack([W4, W6])
    g46 = pl.pallas_call(
        _k3_body, grid=(2 * grid10,),
        in_specs=[
            pl.BlockSpec((1, BM, D), lambda i: (i // grid10, i % grid10, 0)),
            pl.BlockSpec((BM, D), lambda i: (i, 0)),
            pl.BlockSpec((1, 1, D), lambda i: (i // grid10, 0, 0)),
            pl.BlockSpec((NC, BM, DW), lambda i: (0, i % grid10, 0)),
            pl.BlockSpec((1, D, D), lambda i: (i // grid10, 0, 0)),
        ],
        out_specs=pl.BlockSpec((BM, D), lambda i: (i, 0)),
        out_shape=jax.ShapeDtypeStruct((2 * N, D), f32),
    )(pp, g35, b35, degp, w46)

    pp2 = seg_pair(g46, src_pair, dst_pair)

    mu, z = pl.pallas_call(
        _k4_body, grid=(grid10,),
        in_specs=[
            pl.BlockSpec((1, BM, D), lambda i: (0, i, 0)),
            pl.BlockSpec((1, BM, D), lambda i: (1, i, 0)),
            pl.BlockSpec((BM, D), lambda i: (i, 0)),
            pl.BlockSpec((BM, D), lambda i: (grid10 + i, 0)),
            spec_b, spec_b, spec_degp, spec_rows,
        ],
        out_specs=[spec_rows, spec_rows],
        out_shape=[jax.ShapeDtypeStruct((N, D), f32),
                   jax.ShapeDtypeStruct((N, D), f32)],
    )(pp2, pp2, g46, g46, b4.reshape(1, D), b6.reshape(1, D), degp, eps)

    DM, DN = 1024, 2048
    adj = pl.pallas_call(
        _dec_body, grid=(pl.cdiv(N, DM), pl.cdiv(N, DN)),
        in_specs=[
            pl.BlockSpec((DM, D), lambda i, j: (i, 0)),
            pl.BlockSpec((DN, D), lambda i, j: (j, 0)),
        ],
        out_specs=pl.BlockSpec((DM, DN), lambda i, j: (i, j)),
        out_shape=jax.ShapeDtypeStruct((N, N), f32),
    )(z, z)

    return (adj, mu, mu)


# speedup vs baseline: 16.1005x; 1.1467x over previous
import functools

import jax
import jax.numpy as jnp
from jax import lax
from jax.experimental import pallas as pl
from jax.experimental.pallas import tpu as pltpu
from jax.experimental.pallas import tpu_sc as plsc

N = 10000
E = 320000
D = 128
NC = 2
NS = 16
CH = 128
NCHUNK = 2560
E_PAD = NCHUNK * CH
NPAD = 10112
STRIPE = NPAD // NS

BM = 1000


IB = 8


def _segsum_body(d, per_tile, tbl, srcix, dstix, out, src_v, dst_v, rows_v,
                 acc_sh, sem_g, sem_s):
    cid = lax.axis_index("c")
    sid = lax.axis_index("s")
    nbatch = per_tile // IB
    zero16 = jnp.zeros((16,), jnp.float32)
    for r in range(CH):
        for c in range(d // 16):
            rows_v[0, r, pl.ds(c * 16, 16)] = zero16
    row0 = sid * STRIPE
    nfull, rem = STRIPE // CH, STRIPE % CH
    for t in range(nfull):
        pltpu.sync_copy(rows_v.at[0], acc_sh.at[pl.ds(row0 + t * CH, CH)])
    if rem:
        pltpu.sync_copy(rows_v.at[0, pl.ds(0, rem)],
                        acc_sh.at[pl.ds(row0 + nfull * CH, rem)])
    cb = sid * per_tile
    plsc.subcore_barrier()

    def stage(b, parity):
        pltpu.sync_copy(srcix.at[cid, pl.ds(cb + b * IB, IB)],
                        src_v.at[parity])
        pltpu.sync_copy(dstix.at[cid, pl.ds(cb + b * IB, IB)],
                        dst_v.at[parity])

    def gather(parity, j, buf):
        return pltpu.make_async_copy(tbl.at[src_v.at[parity, j]],
                                     rows_v.at[buf], sem_g)

    def scatter(parity, j, buf):
        return pltpu.make_async_copy(rows_v.at[buf],
                                     acc_sh.at[dst_v.at[parity, j]], sem_s)

    stage(0, 0)
    gather(0, 0, 0).start()

    def batch_body(b, carry):
        pb = b % 2
        for j in range(IB):
            buf = j % 2
            k = b * IB + j
            gather(pb, j, buf).wait()
            @pl.when(k > 0)
            def _():
                scatter(pb, j, 1 - buf).wait()
            if j == 0:
                stage((b + 1) % nbatch, 1 - pb)
            nxt_j = (j + 1) % IB
            gather(pb if j + 1 < IB else 1 - pb, nxt_j, 1 - buf).start()
            scatter(pb, j, buf).start(add=True)
        return carry

    lax.fori_loop(0, nbatch, batch_body, 0)
    gather(0, 0, 0).wait()
    scatter(0, IB - 1, (IB - 1) % 2).wait()
    plsc.subcore_barrier()
    pltpu.sync_copy(acc_sh.at[pl.ds(row0, STRIPE)],
                    out.at[cid, pl.ds(row0, STRIPE)])


def _make_segsum(per_core_chunks):
    per_tile = per_core_chunks // NS
    mesh = plsc.VectorSubcoreMesh(core_axis_name="c", subcore_axis_name="s",
                                  num_cores=NC, num_subcores=NS)
    return functools.partial(
        pl.kernel,
        out_type=jax.ShapeDtypeStruct((NC, NPAD, D), jnp.float32),
        mesh=mesh,
        scratch_types=[
            pltpu.VMEM((2, IB, CH), jnp.int32),
            pltpu.VMEM((2, IB, CH), jnp.int32),
            pltpu.VMEM((2, CH, D), jnp.float32),
            pltpu.VMEM_SHARED((NPAD, D), jnp.float32),
            pltpu.SemaphoreType.DMA,
            pltpu.SemaphoreType.DMA,
        ],
    )(functools.partial(_segsum_body, D, per_tile))


DW = 128


def _deg_body(per_tile, dstix, out, dst_v, buf_v, acc_sh, sem_s):
    cid = lax.axis_index("c")
    sid = lax.axis_index("s")
    zero16 = jnp.zeros((16,), jnp.float32)
    for r in range(CH):
        for c in range(DW // 16):
            buf_v[r, pl.ds(c * 16, 16)] = zero16
    row0 = sid * STRIPE
    nfull, rem = STRIPE // CH, STRIPE % CH
    for t in range(nfull):
        pltpu.sync_copy(buf_v, acc_sh.at[pl.ds(row0 + t * CH, CH)])
    if rem:
        pltpu.sync_copy(buf_v.at[pl.ds(0, rem)],
                        acc_sh.at[pl.ds(row0 + nfull * CH, rem)])
    one16 = jnp.ones((16,), jnp.float32)
    for r in range(CH):
        buf_v[r, pl.ds(0, 16)] = one16
    cb = sid * per_tile
    plsc.subcore_barrier()

    def batch_body(b, carry):
        pltpu.sync_copy(dstix.at[cid, pl.ds(cb + b * IB, IB)], dst_v)
        for j in range(IB):
            pltpu.make_async_copy(buf_v, acc_sh.at[dst_v.at[j]],
                                  sem_s).start(add=True)
        for j in range(IB):
            pltpu.make_async_copy(buf_v, acc_sh.at[dst_v.at[j]],
                                  sem_s).wait()
        return carry

    lax.fori_loop(0, per_tile // IB, batch_body, 0)
    plsc.subcore_barrier()
    pltpu.sync_copy(acc_sh.at[pl.ds(row0, STRIPE)],
                    out.at[cid, pl.ds(row0, STRIPE)])


def _make_deg(per_core_chunks):
    per_tile = per_core_chunks // NS
    mesh = plsc.VectorSubcoreMesh(core_axis_name="c", subcore_axis_name="s",
                                  num_cores=NC, num_subcores=NS)
    return functools.partial(
        pl.kernel,
        out_type=jax.ShapeDtypeStruct((NC, NPAD, DW), jnp.float32),
        mesh=mesh,
        scratch_types=[
            pltpu.VMEM((IB, CH), jnp.int32),
            pltpu.VMEM((CH, DW), jnp.float32),
            pltpu.VMEM_SHARED((NPAD, DW), jnp.float32),
            pltpu.SemaphoreType.DMA,
        ],
    )(functools.partial(_deg_body, per_tile))


def _dinv(degp0, degp1):
    return lax.rsqrt(degp0[:, 0:1] + degp1[:, 0:1] + 1.0)


_HI = lax.Precision.HIGHEST


def _k0_body(degp_ref, x_ref, w_ref, g_ref):
    dinv = _dinv(degp_ref[0], degp_ref[1])
    g_ref[...] = dinv * jnp.dot(x_ref[...], w_ref[...],
                                preferred_element_type=jnp.float32,
                                precision=_HI)


def _k1_body(p_ref, g_ref, b_ref, degp_ref, w_ref, o_ref):
    dinv = _dinv(degp_ref[0], degp_ref[1])
    x = jnp.maximum(dinv * (p_ref[0] + p_ref[1] + g_ref[...]) + b_ref[...],
                    0.0)
    o_ref[...] = dinv * jnp.dot(x, w_ref[...],
                                preferred_element_type=jnp.float32,
                                precision=_HI)


def _k2_body(p_ref, g_ref, b_ref, degp_ref, w_ref, o_ref):
    dinv = _dinv(degp_ref[0], degp_ref[1])
    x = jnp.maximum(dinv * (p_ref[0] + p_ref[1] + g_ref[...]) + b_ref[...],
                    0.0)
    o_ref[...] = dinv * jnp.dot(x, w_ref[0],
                                preferred_element_type=jnp.float32,
                                precision=_HI)


def _k3_body(pp_ref, g_ref, b_ref, degp_ref, w_ref, o_ref):
    dinv = _dinv(degp_ref[0], degp_ref[1])
    x = jnp.maximum(dinv * (pp_ref[0] + g_ref[...]) + b_ref[0], 0.0)
    o_ref[...] = dinv * jnp.dot(x, w_ref[0],
                                preferred_element_type=jnp.float32,
                                precision=_HI)


def _k4_body(pa_ref, pb_ref, ga_ref, gb_ref, ba_ref, bb_ref, degp_ref,
             eps_ref, mu_ref, z_ref):
    dinv = _dinv(degp_ref[0], degp_ref[1])
    mu = jnp.maximum(dinv * (pa_ref[0] + ga_ref[...]) + ba_ref[...], 0.0)
    logvar = jnp.maximum(dinv * (pb_ref[0] + gb_ref[...]) + bb_ref[...], 0.0)
    mu_ref[...] = mu
    z_ref[...] = mu + eps_ref[...] * jnp.exp(0.5 * logvar)


def _dec_body(zr_ref, zc_ref, o_ref):
    o_ref[...] = lax.dot_general(
        zr_ref[...].astype(jnp.bfloat16), zc_ref[...].astype(jnp.bfloat16),
        (((1,), (1,)), ((), ())),
        preferred_element_type=jnp.float32)


def kernel(X, edge_index, W0, b0, W1, b1, W2, b2, W3, b3, W4, b4, W5, b5, W6,
           b6):
    f32 = jnp.float32
    src = edge_index[0]
    dst = edge_index[1]
    pad = E_PAD - E
    fill = jnp.arange(pad, dtype=jnp.int32)
    srcp = jnp.concatenate([src, (fill * 131) % N])
    dstp = jnp.concatenate([dst, N + fill % (NPAD - N)])
    src2 = srcp.reshape(NCHUNK, CH)
    dst2 = dstp.reshape(NCHUNK, CH)
    src_trunk = src2.reshape(NC, NCHUNK // NC, CH)
    dst_trunk = dst2.reshape(NC, NCHUNK // NC, CH)
    src_pair = jnp.stack([src2, src2 + N])
    dst_pair = jnp.stack([dst2, dst2])
    eps = jax.random.normal(jax.random.key(42), (N, D), dtype=f32)

    seg_trunk = _make_segsum(NCHUNK // NC)
    seg_pair = _make_segsum(NCHUNK)
    deg_k = _make_deg(NCHUNK // NC)

    degp = deg_k(dst_trunk)

    grid10 = N // BM
    spec_degp = pl.BlockSpec((NC, BM, DW), lambda i: (0, i, 0))
    spec_rows = pl.BlockSpec((BM, D), lambda i: (i, 0))
    spec_w = pl.BlockSpec((D, D), lambda i: (0, 0))
    spec_b = pl.BlockSpec((1, D), lambda i: (0, 0))
    spec_p = pl.BlockSpec((NC, BM, D), lambda i: (0, i, 0))

    b0r, b1r, b2r = b0.reshape(1, D), b1.reshape(1, D), b2.reshape(1, D)

    g = pl.pallas_call(
        _k0_body, grid=(grid10,),
        in_specs=[spec_degp, spec_rows, spec_w],
        out_specs=spec_rows,
        out_shape=jax.ShapeDtypeStruct((N, D), f32),
    )(degp, X, W0)

    for b_i, w_next in ((b0r, W1), (b1r, W2)):
        p = seg_trunk(g, src_trunk, dst_trunk)
        g = pl.pallas_call(
            _k1_body, grid=(grid10,),
            in_specs=[spec_p, spec_rows, spec_b, spec_degp, spec_w],
            out_specs=spec_rows,
            out_shape=jax.ShapeDtypeStruct((N, D), f32),
        )(p, g, b_i, degp, w_next)

    p2 = seg_trunk(g, src_trunk, dst_trunk)
    w35 = jnp.stack([W3, W5])
    g35 = pl.pallas_call(
        _k2_body, grid=(2 * grid10,),
        in_specs=[
            pl.BlockSpec((NC, BM, D), lambda i: (0, i % grid10, 0)),
            pl.BlockSpec((BM, D), lambda i: (i % grid10, 0)),
            spec_b,
            pl.BlockSpec((NC, BM, DW), lambda i: (0, i % grid10, 0)),
            pl.BlockSpec((1, D, D), lambda i: (i // grid10, 0, 0)),
        ],
        out_specs=pl.BlockSpec((BM, D), lambda i: (i, 0)),
        out_shape=jax.ShapeDtypeStruct((2 * N, D), f32),
    )(p2, g, b2r, degp, w35)

    pp = seg_pair(g35, src_pair, dst_pair)

    b35 = jnp.stack([b3.reshape(1, D), b5.reshape(1, D)])
    w46 = jnp.stack([W4, W6])
    g46 = pl.pallas_call(
        _k3_body, grid=(2 * grid10,),
        in_specs=[
            pl.BlockSpec((1, BM, D), lambda i: (i // grid10, i % grid10, 0)),
            pl.BlockSpec((BM, D), lambda i: (i, 0)),
            pl.BlockSpec((1, 1, D), lambda i: (i // grid10, 0, 0)),
            pl.BlockSpec((NC, BM, DW), lambda i: (0, i % grid10, 0)),
            pl.BlockSpec((1, D, D), lambda i: (i // grid10, 0, 0)),
        ],
        out_specs=pl.BlockSpec((BM, D), lambda i: (i, 0)),
        out_shape=jax.ShapeDtypeStruct((2 * N, D), f32),
    )(pp, g35, b35, degp, w46)

    pp2 = seg_pair(g46, src_pair, dst_pair)

    mu, z = pl.pallas_call(
        _k4_body, grid=(grid10,),
        in_specs=[
            pl.BlockSpec((1, BM, D), lambda i: (0, i, 0)),
            pl.BlockSpec((1, BM, D), lambda i: (1, i, 0)),
            pl.BlockSpec((BM, D), lambda i: (i, 0)),
            pl.BlockSpec((BM, D), lambda i: (grid10 + i, 0)),
            spec_b, spec_b, spec_degp, spec_rows,
        ],
        out_specs=[spec_rows, spec_rows],
        out_shape=[jax.ShapeDtypeStruct((N, D), f32),
                   jax.ShapeDtypeStruct((N, D), f32)],
    )(pp2, pp2, g46, g46, b4.reshape(1, D), b6.reshape(1, D), degp, eps)

    DM, DN = 1024, 2048
    adj = pl.pallas_call(
        _dec_body, grid=(pl.cdiv(N, DM), pl.cdiv(N, DN)),
        in_specs=[
            pl.BlockSpec((DM, D), lambda i, j: (i, 0)),
            pl.BlockSpec((DN, D), lambda i, j: (j, 0)),
        ],
        out_specs=pl.BlockSpec((DM, DN), lambda i, j: (i, j)),
        out_shape=jax.ShapeDtypeStruct((N, N), f32),
    )(z, z)

    return (adj, mu, mu)


# speedup vs baseline: 16.9456x; 1.0525x over previous
import functools

import jax
import jax.numpy as jnp
from jax import lax
from jax.experimental import pallas as pl
from jax.experimental.pallas import tpu as pltpu
from jax.experimental.pallas import tpu_sc as plsc

N = 10000
E = 320000
D = 128
NC = 2
NS = 16
CH = 64
NCHUNK = 5120
NBUF = 4
E_PAD = NCHUNK * CH
NPAD = 10112
STRIPE = NPAD // NS

BM = 1000


IB = 8


def _segsum_body(d, per_tile, tbl, srcix, dstix, out, src_v, dst_v, rows_v,
                 acc_sh, sem_g, sem_s):
    cid = lax.axis_index("c")
    sid = lax.axis_index("s")
    nbatch = per_tile // IB
    zero16 = jnp.zeros((16,), jnp.float32)
    for r in range(CH):
        for c in range(d // 16):
            rows_v[0, r, pl.ds(c * 16, 16)] = zero16
    row0 = sid * STRIPE
    nfull, rem = STRIPE // CH, STRIPE % CH
    for t in range(nfull):
        pltpu.sync_copy(rows_v.at[0], acc_sh.at[pl.ds(row0 + t * CH, CH)])
    if rem:
        pltpu.sync_copy(rows_v.at[0, pl.ds(0, rem)],
                        acc_sh.at[pl.ds(row0 + nfull * CH, rem)])
    cb = sid * per_tile
    plsc.subcore_barrier()

    def stage(b, parity):
        pltpu.sync_copy(srcix.at[cid, pl.ds(cb + b * IB, IB)],
                        src_v.at[parity])
        pltpu.sync_copy(dstix.at[cid, pl.ds(cb + b * IB, IB)],
                        dst_v.at[parity])

    def gather(parity, j, buf):
        return pltpu.make_async_copy(tbl.at[src_v.at[parity, j]],
                                     rows_v.at[buf], sem_g)

    def scatter(parity, j, buf):
        return pltpu.make_async_copy(rows_v.at[buf],
                                     acc_sh.at[dst_v.at[parity, j]], sem_s)

    stage(0, 0)
    gather(0, 0, 0).start()
    gather(0, 1, 1).start()

    def batch_body(b, carry):
        pb = b % 2
        for j in range(IB):
            buf = j % NBUF
            k = b * IB + j
            gather(pb, j, buf).wait()
            scatter(pb, j, buf).start(add=True)
            @pl.when(k >= 2)
            def _():
                scatter(pb, j, (buf + 2) % NBUF).wait()
            if j == 2:
                stage((b + 1) % nbatch, 1 - pb)
            nj = (j + 2) % IB
            gather(pb if j + 2 < IB else 1 - pb, nj,
                   (buf + 2) % NBUF).start()
        return carry

    lax.fori_loop(0, nbatch, batch_body, 0)
    gather(0, 0, 0).wait()
    gather(0, 1, 1).wait()
    scatter(0, IB - 2, (IB - 2) % NBUF).wait()
    scatter(0, IB - 1, (IB - 1) % NBUF).wait()
    plsc.subcore_barrier()
    pltpu.sync_copy(acc_sh.at[pl.ds(row0, STRIPE)],
                    out.at[cid, pl.ds(row0, STRIPE)])


def _make_segsum(per_core_chunks):
    per_tile = per_core_chunks // NS
    mesh = plsc.VectorSubcoreMesh(core_axis_name="c", subcore_axis_name="s",
                                  num_cores=NC, num_subcores=NS)
    return functools.partial(
        pl.kernel,
        out_type=jax.ShapeDtypeStruct((NC, NPAD, D), jnp.float32),
        mesh=mesh,
        scratch_types=[
            pltpu.VMEM((2, IB, CH), jnp.int32),
            pltpu.VMEM((2, IB, CH), jnp.int32),
            pltpu.VMEM((NBUF, CH, D), jnp.float32),
            pltpu.VMEM_SHARED((NPAD, D), jnp.float32),
            pltpu.SemaphoreType.DMA,
            pltpu.SemaphoreType.DMA,
        ],
    )(functools.partial(_segsum_body, D, per_tile))


DW = 128


def _deg_body(per_tile, dstix, out, dst_v, buf_v, acc_sh, sem_s):
    cid = lax.axis_index("c")
    sid = lax.axis_index("s")
    zero16 = jnp.zeros((16,), jnp.float32)
    for r in range(CH):
        for c in range(DW // 16):
            buf_v[r, pl.ds(c * 16, 16)] = zero16
    row0 = sid * STRIPE
    nfull, rem = STRIPE // CH, STRIPE % CH
    for t in range(nfull):
        pltpu.sync_copy(buf_v, acc_sh.at[pl.ds(row0 + t * CH, CH)])
    if rem:
        pltpu.sync_copy(buf_v.at[pl.ds(0, rem)],
                        acc_sh.at[pl.ds(row0 + nfull * CH, rem)])
    one16 = jnp.ones((16,), jnp.float32)
    for r in range(CH):
        buf_v[r, pl.ds(0, 16)] = one16
    cb = sid * per_tile
    plsc.subcore_barrier()

    def batch_body(b, carry):
        pltpu.sync_copy(dstix.at[cid, pl.ds(cb + b * IB, IB)], dst_v)
        for j in range(IB):
            pltpu.make_async_copy(buf_v, acc_sh.at[dst_v.at[j]],
                                  sem_s).start(add=True)
        for j in range(IB):
            pltpu.make_async_copy(buf_v, acc_sh.at[dst_v.at[j]],
                                  sem_s).wait()
        return carry

    lax.fori_loop(0, per_tile // IB, batch_body, 0)
    plsc.subcore_barrier()
    pltpu.sync_copy(acc_sh.at[pl.ds(row0, STRIPE)],
                    out.at[cid, pl.ds(row0, STRIPE)])


def _make_deg(per_core_chunks):
    per_tile = per_core_chunks // NS
    mesh = plsc.VectorSubcoreMesh(core_axis_name="c", subcore_axis_name="s",
                                  num_cores=NC, num_subcores=NS)
    return functools.partial(
        pl.kernel,
        out_type=jax.ShapeDtypeStruct((NC, NPAD, DW), jnp.float32),
        mesh=mesh,
        scratch_types=[
            pltpu.VMEM((IB, CH), jnp.int32),
            pltpu.VMEM((CH, DW), jnp.float32),
            pltpu.VMEM_SHARED((NPAD, DW), jnp.float32),
            pltpu.SemaphoreType.DMA,
        ],
    )(functools.partial(_deg_body, per_tile))


def _dinv(degp0, degp1):
    return lax.rsqrt(degp0[:, 0:1] + degp1[:, 0:1] + 1.0)


_HI = lax.Precision.HIGHEST


def _k0_body(degp_ref, x_ref, w_ref, g_ref):
    dinv = _dinv(degp_ref[0], degp_ref[1])
    g_ref[...] = dinv * jnp.dot(x_ref[...], w_ref[...],
                                preferred_element_type=jnp.float32,
                                precision=_HI)


def _k1_body(p_ref, g_ref, b_ref, degp_ref, w_ref, o_ref):
    dinv = _dinv(degp_ref[0], degp_ref[1])
    x = jnp.maximum(dinv * (p_ref[0] + p_ref[1] + g_ref[...]) + b_ref[...],
                    0.0)
    o_ref[...] = dinv * jnp.dot(x, w_ref[...],
                                preferred_element_type=jnp.float32,
                                precision=_HI)


def _k2_body(p_ref, g_ref, b_ref, degp_ref, w_ref, o_ref):
    dinv = _dinv(degp_ref[0], degp_ref[1])
    x = jnp.maximum(dinv * (p_ref[0] + p_ref[1] + g_ref[...]) + b_ref[...],
                    0.0)
    o_ref[...] = dinv * jnp.dot(x, w_ref[0],
                                preferred_element_type=jnp.float32,
                                precision=_HI)


def _k3_body(pp_ref, g_ref, b_ref, degp_ref, w_ref, o_ref):
    dinv = _dinv(degp_ref[0], degp_ref[1])
    x = jnp.maximum(dinv * (pp_ref[0] + g_ref[...]) + b_ref[0], 0.0)
    o_ref[...] = dinv * jnp.dot(x, w_ref[0],
                                preferred_element_type=jnp.float32,
                                precision=_HI)


def _k4_body(pa_ref, pb_ref, ga_ref, gb_ref, ba_ref, bb_ref, degp_ref,
             eps_ref, mu_ref, z_ref):
    dinv = _dinv(degp_ref[0], degp_ref[1])
    mu = jnp.maximum(dinv * (pa_ref[0] + ga_ref[...]) + ba_ref[...], 0.0)
    logvar = jnp.maximum(dinv * (pb_ref[0] + gb_ref[...]) + bb_ref[...], 0.0)
    mu_ref[...] = mu
    z_ref[...] = mu + eps_ref[...] * jnp.exp(0.5 * logvar)


def _dec_body(zr_ref, zc_ref, o_ref):
    o_ref[...] = lax.dot_general(
        zr_ref[...].astype(jnp.bfloat16), zc_ref[...].astype(jnp.bfloat16),
        (((1,), (1,)), ((), ())),
        preferred_element_type=jnp.float32)


def kernel(X, edge_index, W0, b0, W1, b1, W2, b2, W3, b3, W4, b4, W5, b5, W6,
           b6):
    f32 = jnp.float32
    src = edge_index[0]
    dst = edge_index[1]
    pad = E_PAD - E
    fill = jnp.arange(pad, dtype=jnp.int32)
    srcp = jnp.concatenate([src, (fill * 131) % N])
    dstp = jnp.concatenate([dst, N + fill % (NPAD - N)])
    src2 = srcp.reshape(NCHUNK, CH)
    dst2 = dstp.reshape(NCHUNK, CH)
    src_trunk = src2.reshape(NC, NCHUNK // NC, CH)
    dst_trunk = dst2.reshape(NC, NCHUNK // NC, CH)
    src_pair = jnp.stack([src2, src2 + N])
    dst_pair = jnp.stack([dst2, dst2])
    eps = jax.random.normal(jax.random.key(42), (N, D), dtype=f32)

    seg_trunk = _make_segsum(NCHUNK // NC)
    seg_pair = _make_segsum(NCHUNK)
    deg_k = _make_deg(NCHUNK // NC)

    degp = deg_k(dst_trunk)

    grid10 = N // BM
    spec_degp = pl.BlockSpec((NC, BM, DW), lambda i: (0, i, 0))
    spec_rows = pl.BlockSpec((BM, D), lambda i: (i, 0))
    spec_w = pl.BlockSpec((D, D), lambda i: (0, 0))
    spec_b = pl.BlockSpec((1, D), lambda i: (0, 0))
    spec_p = pl.BlockSpec((NC, BM, D), lambda i: (0, i, 0))

    b0r, b1r, b2r = b0.reshape(1, D), b1.reshape(1, D), b2.reshape(1, D)

    g = pl.pallas_call(
        _k0_body, grid=(grid10,),
        in_specs=[spec_degp, spec_rows, spec_w],
        out_specs=spec_rows,
        out_shape=jax.ShapeDtypeStruct((N, D), f32),
    )(degp, X, W0)

    for b_i, w_next in ((b0r, W1), (b1r, W2)):
        p = seg_trunk(g, src_trunk, dst_trunk)
        g = pl.pallas_call(
            _k1_body, grid=(grid10,),
            in_specs=[spec_p, spec_rows, spec_b, spec_degp, spec_w],
            out_specs=spec_rows,
            out_shape=jax.ShapeDtypeStruct((N, D), f32),
        )(p, g, b_i, degp, w_next)

    p2 = seg_trunk(g, src_trunk, dst_trunk)
    w35 = jnp.stack([W3, W5])
    g35 = pl.pallas_call(
        _k2_body, grid=(2 * grid10,),
        in_specs=[
            pl.BlockSpec((NC, BM, D), lambda i: (0, i % grid10, 0)),
            pl.BlockSpec((BM, D), lambda i: (i % grid10, 0)),
            spec_b,
            pl.BlockSpec((NC, BM, DW), lambda i: (0, i % grid10, 0)),
            pl.BlockSpec((1, D, D), lambda i: (i // grid10, 0, 0)),
        ],
        out_specs=pl.BlockSpec((BM, D), lambda i: (i, 0)),
        out_shape=jax.ShapeDtypeStruct((2 * N, D), f32),
    )(p2, g, b2r, degp, w35)

    pp = seg_pair(g35, src_pair, dst_pair)

    b35 = jnp.stack([b3.reshape(1, D), b5.reshape(1, D)])
    w46 = jnp.stack([W4, W6])
    g46 = pl.pallas_call(
        _k3_body, grid=(2 * grid10,),
        in_specs=[
            pl.BlockSpec((1, BM, D), lambda i: (i // grid10, i % grid10, 0)),
            pl.BlockSpec((BM, D), lambda i: (i, 0)),
            pl.BlockSpec((1, 1, D), lambda i: (i // grid10, 0, 0)),
            pl.BlockSpec((NC, BM, DW), lambda i: (0, i % grid10, 0)),
            pl.BlockSpec((1, D, D), lambda i: (i // grid10, 0, 0)),
        ],
        out_specs=pl.BlockSpec((BM, D), lambda i: (i, 0)),
        out_shape=jax.ShapeDtypeStruct((2 * N, D), f32),
    )(pp, g35, b35, degp, w46)

    pp2 = seg_pair(g46, src_pair, dst_pair)

    mu, z = pl.pallas_call(
        _k4_body, grid=(grid10,),
        in_specs=[
            pl.BlockSpec((1, BM, D), lambda i: (0, i, 0)),
            pl.BlockSpec((1, BM, D), lambda i: (1, i, 0)),
            pl.BlockSpec((BM, D), lambda i: (i, 0)),
            pl.BlockSpec((BM, D), lambda i: (grid10 + i, 0)),
            spec_b, spec_b, spec_degp, spec_rows,
        ],
        out_specs=[spec_rows, spec_rows],
        out_shape=[jax.ShapeDtypeStruct((N, D), f32),
                   jax.ShapeDtypeStruct((N, D), f32)],
    )(pp2, pp2, g46, g46, b4.reshape(1, D), b6.reshape(1, D), degp, eps)

    DM, DN = 1024, 2048
    adj = pl.pallas_call(
        _dec_body, grid=(pl.cdiv(N, DM), pl.cdiv(N, DN)),
        in_specs=[
            pl.BlockSpec((DM, D), lambda i, j: (i, 0)),
            pl.BlockSpec((DN, D), lambda i, j: (j, 0)),
        ],
        out_specs=pl.BlockSpec((DM, DN), lambda i, j: (i, j)),
        out_shape=jax.ShapeDtypeStruct((N, N), f32),
    )(z, z)

    return (adj, mu, mu)


# speedup vs baseline: 17.2646x; 1.0188x over previous
import functools

import jax
import jax.numpy as jnp
from jax import lax
from jax.experimental import pallas as pl
from jax.experimental.pallas import tpu as pltpu
from jax.experimental.pallas import tpu_sc as plsc

N = 10000
E = 320000
D = 128
NC = 2
NS = 16
CH = 64
NCHUNK = 5120
NBUF = 4
E_PAD = NCHUNK * CH
NPAD = 10112
STRIPE = NPAD // NS

BM = 1000


IB = 8


def _segsum_body(d, per_tile, tbl, srcix, dstix, out, src_v, dst_v, rows_v,
                 acc_sh, sem_g, sem_s):
    cid = lax.axis_index("c")
    sid = lax.axis_index("s")
    nbatch = per_tile // IB
    zero16 = jnp.zeros((16,), jnp.float32)
    for r in range(CH):
        for c in range(d // 16):
            rows_v[0, r, pl.ds(c * 16, 16)] = zero16
    row0 = sid * STRIPE
    nfull, rem = STRIPE // CH, STRIPE % CH
    for t in range(nfull):
        pltpu.sync_copy(rows_v.at[0], acc_sh.at[pl.ds(row0 + t * CH, CH)])
    if rem:
        pltpu.sync_copy(rows_v.at[0, pl.ds(0, rem)],
                        acc_sh.at[pl.ds(row0 + nfull * CH, rem)])
    cb = sid * per_tile
    plsc.subcore_barrier()

    def stage(b, parity):
        pltpu.sync_copy(srcix.at[cid, pl.ds(cb + b * IB, IB)],
                        src_v.at[parity])
        pltpu.sync_copy(dstix.at[cid, pl.ds(cb + b * IB, IB)],
                        dst_v.at[parity])

    def gather(parity, j, buf):
        return pltpu.make_async_copy(tbl.at[src_v.at[parity, j]],
                                     rows_v.at[buf], sem_g)

    def scatter(parity, j, buf):
        return pltpu.make_async_copy(rows_v.at[buf],
                                     acc_sh.at[dst_v.at[parity, j]], sem_s)

    stage(0, 0)
    gather(0, 0, 0).start()
    gather(0, 1, 1).start()

    def batch_body(b, carry):
        pb = b % 2
        for j in range(IB):
            buf = j % NBUF
            k = b * IB + j
            gather(pb, j, buf).wait()
            scatter(pb, j, buf).start(add=True)
            @pl.when(k >= 2)
            def _():
                scatter(pb, j, (buf + 2) % NBUF).wait()
            if j == 2:
                stage((b + 1) % nbatch, 1 - pb)
            nj = (j + 2) % IB
            gather(pb if j + 2 < IB else 1 - pb, nj,
                   (buf + 2) % NBUF).start()
        return carry

    lax.fori_loop(0, nbatch, batch_body, 0)
    gather(0, 0, 0).wait()
    gather(0, 1, 1).wait()
    scatter(0, IB - 2, (IB - 2) % NBUF).wait()
    scatter(0, IB - 1, (IB - 1) % NBUF).wait()
    plsc.subcore_barrier()
    pltpu.sync_copy(acc_sh.at[pl.ds(row0, STRIPE)],
                    out.at[cid, pl.ds(row0, STRIPE)])


def _make_segsum(per_core_chunks):
    per_tile = per_core_chunks // NS
    mesh = plsc.VectorSubcoreMesh(core_axis_name="c", subcore_axis_name="s",
                                  num_cores=NC, num_subcores=NS)
    return functools.partial(
        pl.kernel,
        out_type=jax.ShapeDtypeStruct((NC, NPAD, D), jnp.float32),
        mesh=mesh,
        scratch_types=[
            pltpu.VMEM((2, IB, CH), jnp.int32),
            pltpu.VMEM((2, IB, CH), jnp.int32),
            pltpu.VMEM((NBUF, CH, D), jnp.float32),
            pltpu.VMEM_SHARED((NPAD, D), jnp.float32),
            pltpu.SemaphoreType.DMA,
            pltpu.SemaphoreType.DMA,
        ],
    )(functools.partial(_segsum_body, D, per_tile))


DW = 128


def _deg_body(per_tile, dstix, out, dst_v, buf_v, acc_sh, sem_s):
    cid = lax.axis_index("c")
    sid = lax.axis_index("s")
    zero16 = jnp.zeros((16,), jnp.float32)
    for r in range(CH):
        for c in range(DW // 16):
            buf_v[r, pl.ds(c * 16, 16)] = zero16
    row0 = sid * STRIPE
    nfull, rem = STRIPE // CH, STRIPE % CH
    for t in range(nfull):
        pltpu.sync_copy(buf_v, acc_sh.at[pl.ds(row0 + t * CH, CH)])
    if rem:
        pltpu.sync_copy(buf_v.at[pl.ds(0, rem)],
                        acc_sh.at[pl.ds(row0 + nfull * CH, rem)])
    one16 = jnp.ones((16,), jnp.float32)
    for r in range(CH):
        buf_v[r, pl.ds(0, 16)] = one16
    cb = sid * per_tile
    plsc.subcore_barrier()

    def batch_body(b, carry):
        pltpu.sync_copy(dstix.at[cid, pl.ds(cb + b * IB, IB)], dst_v)
        for j in range(IB):
            pltpu.make_async_copy(buf_v, acc_sh.at[dst_v.at[j]],
                                  sem_s).start(add=True)
        for j in range(IB):
            pltpu.make_async_copy(buf_v, acc_sh.at[dst_v.at[j]],
                                  sem_s).wait()
        return carry

    lax.fori_loop(0, per_tile // IB, batch_body, 0)
    plsc.subcore_barrier()
    pltpu.sync_copy(acc_sh.at[pl.ds(row0, STRIPE)],
                    out.at[cid, pl.ds(row0, STRIPE)])


def _make_deg(per_core_chunks):
    per_tile = per_core_chunks // NS
    mesh = plsc.VectorSubcoreMesh(core_axis_name="c", subcore_axis_name="s",
                                  num_cores=NC, num_subcores=NS)
    return functools.partial(
        pl.kernel,
        out_type=jax.ShapeDtypeStruct((NC, NPAD, DW), jnp.float32),
        mesh=mesh,
        scratch_types=[
            pltpu.VMEM((IB, CH), jnp.int32),
            pltpu.VMEM((CH, DW), jnp.float32),
            pltpu.VMEM_SHARED((NPAD, DW), jnp.float32),
            pltpu.SemaphoreType.DMA,
        ],
    )(functools.partial(_deg_body, per_tile))


def _dinv(degp0, degp1):
    return lax.rsqrt(degp0[:, 0:1] + degp1[:, 0:1] + 1.0)


def _k0_body(degp_ref, x_ref, w_ref, g_ref):
    dinv = _dinv(degp_ref[0], degp_ref[1])
    g_ref[...] = dinv * jnp.dot(x_ref[...], w_ref[...],
                                preferred_element_type=jnp.float32)


def _k1_body(p_ref, g_ref, b_ref, degp_ref, w_ref, o_ref):
    dinv = _dinv(degp_ref[0], degp_ref[1])
    x = jnp.maximum(dinv * (p_ref[0] + p_ref[1] + g_ref[...]) + b_ref[...],
                    0.0)
    o_ref[...] = dinv * jnp.dot(x, w_ref[...],
                                preferred_element_type=jnp.float32)


def _k2_body(p_ref, g_ref, b_ref, degp_ref, w_ref, o_ref):
    dinv = _dinv(degp_ref[0], degp_ref[1])
    x = jnp.maximum(dinv * (p_ref[0] + p_ref[1] + g_ref[...]) + b_ref[...],
                    0.0)
    o_ref[...] = dinv * jnp.dot(x, w_ref[0],
                                preferred_element_type=jnp.float32)


def _k3_body(pp_ref, g_ref, b_ref, degp_ref, w_ref, o_ref):
    dinv = _dinv(degp_ref[0], degp_ref[1])
    x = jnp.maximum(dinv * (pp_ref[0] + g_ref[...]) + b_ref[0], 0.0)
    o_ref[...] = dinv * jnp.dot(x, w_ref[0],
                                preferred_element_type=jnp.float32)


def _k4_body(pa_ref, pb_ref, ga_ref, gb_ref, ba_ref, bb_ref, degp_ref,
             eps_ref, mu_ref, z_ref):
    dinv = _dinv(degp_ref[0], degp_ref[1])
    mu = jnp.maximum(dinv * (pa_ref[0] + ga_ref[...]) + ba_ref[...], 0.0)
    logvar = jnp.maximum(dinv * (pb_ref[0] + gb_ref[...]) + bb_ref[...], 0.0)
    mu_ref[...] = mu
    z_ref[...] = mu + eps_ref[...] * jnp.exp(0.5 * logvar)


def _dec_body(zr_ref, zc_ref, o_ref):
    o_ref[...] = lax.dot_general(
        zr_ref[...].astype(jnp.bfloat16), zc_ref[...].astype(jnp.bfloat16),
        (((1,), (1,)), ((), ())),
        preferred_element_type=jnp.float32)


def kernel(X, edge_index, W0, b0, W1, b1, W2, b2, W3, b3, W4, b4, W5, b5, W6,
           b6):
    f32 = jnp.float32
    src = edge_index[0]
    dst = edge_index[1]
    pad = E_PAD - E
    fill = jnp.arange(pad, dtype=jnp.int32)
    srcp = jnp.concatenate([src, (fill * 131) % N])
    dstp = jnp.concatenate([dst, N + fill % (NPAD - N)])
    src2 = srcp.reshape(NCHUNK, CH)
    dst2 = dstp.reshape(NCHUNK, CH)
    src_trunk = src2.reshape(NC, NCHUNK // NC, CH)
    dst_trunk = dst2.reshape(NC, NCHUNK // NC, CH)
    src_pair = jnp.stack([src2, src2 + N])
    dst_pair = jnp.stack([dst2, dst2])
    eps = jax.random.normal(jax.random.key(42), (N, D), dtype=f32)

    seg_trunk = _make_segsum(NCHUNK // NC)
    seg_pair = _make_segsum(NCHUNK)
    deg_k = _make_deg(NCHUNK // NC)

    degp = deg_k(dst_trunk)

    grid10 = N // BM
    spec_degp = pl.BlockSpec((NC, BM, DW), lambda i: (0, i, 0))
    spec_rows = pl.BlockSpec((BM, D), lambda i: (i, 0))
    spec_w = pl.BlockSpec((D, D), lambda i: (0, 0))
    spec_b = pl.BlockSpec((1, D), lambda i: (0, 0))
    spec_p = pl.BlockSpec((NC, BM, D), lambda i: (0, i, 0))

    b0r, b1r, b2r = b0.reshape(1, D), b1.reshape(1, D), b2.reshape(1, D)

    g = pl.pallas_call(
        _k0_body, grid=(grid10,),
        in_specs=[spec_degp, spec_rows, spec_w],
        out_specs=spec_rows,
        out_shape=jax.ShapeDtypeStruct((N, D), f32),
    )(degp, X, W0)

    for b_i, w_next in ((b0r, W1), (b1r, W2)):
        p = seg_trunk(g, src_trunk, dst_trunk)
        g = pl.pallas_call(
            _k1_body, grid=(grid10,),
            in_specs=[spec_p, spec_rows, spec_b, spec_degp, spec_w],
            out_specs=spec_rows,
            out_shape=jax.ShapeDtypeStruct((N, D), f32),
        )(p, g, b_i, degp, w_next)

    p2 = seg_trunk(g, src_trunk, dst_trunk)
    w35 = jnp.stack([W3, W5])
    g35 = pl.pallas_call(
        _k2_body, grid=(2 * grid10,),
        in_specs=[
            pl.BlockSpec((NC, BM, D), lambda i: (0, i % grid10, 0)),
            pl.BlockSpec((BM, D), lambda i: (i % grid10, 0)),
            spec_b,
            pl.BlockSpec((NC, BM, DW), lambda i: (0, i % grid10, 0)),
            pl.BlockSpec((1, D, D), lambda i: (i // grid10, 0, 0)),
        ],
        out_specs=pl.BlockSpec((BM, D), lambda i: (i, 0)),
        out_shape=jax.ShapeDtypeStruct((2 * N, D), f32),
    )(p2, g, b2r, degp, w35)

    pp = seg_pair(g35, src_pair, dst_pair)

    b35 = jnp.stack([b3.reshape(1, D), b5.reshape(1, D)])
    w46 = jnp.stack([W4, W6])
    g46 = pl.pallas_call(
        _k3_body, grid=(2 * grid10,),
        in_specs=[
            pl.BlockSpec((1, BM, D), lambda i: (i // grid10, i % grid10, 0)),
            pl.BlockSpec((BM, D), lambda i: (i, 0)),
            pl.BlockSpec((1, 1, D), lambda i: (i // grid10, 0, 0)),
            pl.BlockSpec((NC, BM, DW), lambda i: (0, i % grid10, 0)),
            pl.BlockSpec((1, D, D), lambda i: (i // grid10, 0, 0)),
        ],
        out_specs=pl.BlockSpec((BM, D), lambda i: (i, 0)),
        out_shape=jax.ShapeDtypeStruct((2 * N, D), f32),
    )(pp, g35, b35, degp, w46)

    pp2 = seg_pair(g46, src_pair, dst_pair)

    mu, z = pl.pallas_call(
        _k4_body, grid=(grid10,),
        in_specs=[
            pl.BlockSpec((1, BM, D), lambda i: (0, i, 0)),
            pl.BlockSpec((1, BM, D), lambda i: (1, i, 0)),
            pl.BlockSpec((BM, D), lambda i: (i, 0)),
            pl.BlockSpec((BM, D), lambda i: (grid10 + i, 0)),
            spec_b, spec_b, spec_degp, spec_rows,
        ],
        out_specs=[spec_rows, spec_rows],
        out_shape=[jax.ShapeDtypeStruct((N, D), f32),
                   jax.ShapeDtypeStruct((N, D), f32)],
    )(pp2, pp2, g46, g46, b4.reshape(1, D), b6.reshape(1, D), degp, eps)

    DM, DN = 1024, 2048
    adj = pl.pallas_call(
        _dec_body, grid=(pl.cdiv(N, DM), pl.cdiv(N, DN)),
        in_specs=[
            pl.BlockSpec((DM, D), lambda i, j: (i, 0)),
            pl.BlockSpec((DN, D), lambda i, j: (j, 0)),
        ],
        out_specs=pl.BlockSpec((DM, DN), lambda i, j: (i, j)),
        out_shape=jax.ShapeDtypeStruct((N, N), f32),
    )(z, z)

    return (adj, mu, mu)


# speedup vs baseline: 17.4454x; 1.0105x over previous
import functools

import jax
import jax.numpy as jnp
from jax import lax
from jax.experimental import pallas as pl
from jax.experimental.pallas import tpu as pltpu
from jax.experimental.pallas import tpu_sc as plsc

N = 10000
E = 320000
D = 128
NC = 2
NS = 16
CH = 64
NCHUNK = 5120
NBUF = 4
E_PAD = NCHUNK * CH
NPAD = 10112
STRIPE = NPAD // NS

BM = 1000


IB = 8


def _segsum_body(d, per_tile, tbl, srcix, dstix, out, src_v, dst_v, rows_v,
                 acc_sh, sem_g, sem_s):
    cid = lax.axis_index("c")
    sid = lax.axis_index("s")
    nbatch = per_tile // IB
    zero16 = jnp.zeros((16,), jnp.float32)
    for r in range(CH):
        for c in range(d // 16):
            rows_v[0, r, pl.ds(c * 16, 16)] = zero16
    row0 = sid * STRIPE
    nfull, rem = STRIPE // CH, STRIPE % CH
    for t in range(nfull):
        pltpu.sync_copy(rows_v.at[0], acc_sh.at[pl.ds(row0 + t * CH, CH)])
    if rem:
        pltpu.sync_copy(rows_v.at[0, pl.ds(0, rem)],
                        acc_sh.at[pl.ds(row0 + nfull * CH, rem)])
    cb = sid * per_tile
    plsc.subcore_barrier()

    def stage(b, parity):
        pltpu.sync_copy(srcix.at[cid, pl.ds(cb + b * IB, IB)],
                        src_v.at[parity])
        pltpu.sync_copy(dstix.at[cid, pl.ds(cb + b * IB, IB)],
                        dst_v.at[parity])

    def gather(parity, j, buf):
        return pltpu.make_async_copy(tbl.at[src_v.at[parity, j]],
                                     rows_v.at[buf], sem_g)

    def scatter(parity, j, buf):
        return pltpu.make_async_copy(rows_v.at[buf],
                                     acc_sh.at[dst_v.at[parity, j]], sem_s)

    stage(0, 0)
    gather(0, 0, 0).start()
    gather(0, 1, 1).start()

    def batch_body(b, carry):
        pb = b % 2
        for j in range(IB):
            buf = j % NBUF
            k = b * IB + j
            gather(pb, j, buf).wait()
            scatter(pb, j, buf).start(add=True)
            @pl.when(k >= 2)
            def _():
                scatter(pb, j, (buf + 2) % NBUF).wait()
            if j == 2:
                stage((b + 1) % nbatch, 1 - pb)
            nj = (j + 2) % IB
            gather(pb if j + 2 < IB else 1 - pb, nj,
                   (buf + 2) % NBUF).start()
        return carry

    lax.fori_loop(0, nbatch, batch_body, 0)
    gather(0, 0, 0).wait()
    gather(0, 1, 1).wait()
    scatter(0, IB - 2, (IB - 2) % NBUF).wait()
    scatter(0, IB - 1, (IB - 1) % NBUF).wait()
    plsc.subcore_barrier()
    pltpu.sync_copy(acc_sh.at[pl.ds(row0, STRIPE)],
                    out.at[cid, pl.ds(row0, STRIPE)])


def _make_segsum(per_core_chunks):
    per_tile = per_core_chunks // NS
    mesh = plsc.VectorSubcoreMesh(core_axis_name="c", subcore_axis_name="s",
                                  num_cores=NC, num_subcores=NS)
    return functools.partial(
        pl.kernel,
        out_type=jax.ShapeDtypeStruct((NC, NPAD, D), jnp.float32),
        mesh=mesh,
        scratch_types=[
            pltpu.VMEM((2, IB, CH), jnp.int32),
            pltpu.VMEM((2, IB, CH), jnp.int32),
            pltpu.VMEM((NBUF, CH, D), jnp.float32),
            pltpu.VMEM_SHARED((NPAD, D), jnp.float32),
            pltpu.SemaphoreType.DMA,
            pltpu.SemaphoreType.DMA,
        ],
    )(functools.partial(_segsum_body, D, per_tile))


DW = 128


def _deg_body(per_tile, dstix, out, dst_v, buf_v, acc_sh, sem_s):
    cid = lax.axis_index("c")
    sid = lax.axis_index("s")
    zero16 = jnp.zeros((16,), jnp.float32)
    for r in range(CH):
        for c in range(DW // 16):
            buf_v[r, pl.ds(c * 16, 16)] = zero16
    row0 = sid * STRIPE
    nfull, rem = STRIPE // CH, STRIPE % CH
    for t in range(nfull):
        pltpu.sync_copy(buf_v, acc_sh.at[pl.ds(row0 + t * CH, CH)])
    if rem:
        pltpu.sync_copy(buf_v.at[pl.ds(0, rem)],
                        acc_sh.at[pl.ds(row0 + nfull * CH, rem)])
    one16 = jnp.ones((16,), jnp.float32)
    for r in range(CH):
        buf_v[r, pl.ds(0, 16)] = one16
    cb = sid * per_tile
    plsc.subcore_barrier()

    def batch_body(b, carry):
        pltpu.sync_copy(dstix.at[cid, pl.ds(cb + b * IB, IB)], dst_v)
        for j in range(IB):
            pltpu.make_async_copy(buf_v, acc_sh.at[dst_v.at[j]],
                                  sem_s).start(add=True)
        for j in range(IB):
            pltpu.make_async_copy(buf_v, acc_sh.at[dst_v.at[j]],
                                  sem_s).wait()
        return carry

    lax.fori_loop(0, per_tile // IB, batch_body, 0)
    plsc.subcore_barrier()
    pltpu.sync_copy(acc_sh.at[pl.ds(row0, STRIPE)],
                    out.at[cid, pl.ds(row0, STRIPE)])


def _make_deg(per_core_chunks):
    per_tile = per_core_chunks // NS
    mesh = plsc.VectorSubcoreMesh(core_axis_name="c", subcore_axis_name="s",
                                  num_cores=NC, num_subcores=NS)
    return functools.partial(
        pl.kernel,
        out_type=jax.ShapeDtypeStruct((NC, NPAD, DW), jnp.float32),
        mesh=mesh,
        scratch_types=[
            pltpu.VMEM((IB, CH), jnp.int32),
            pltpu.VMEM((CH, DW), jnp.float32),
            pltpu.VMEM_SHARED((NPAD, DW), jnp.float32),
            pltpu.SemaphoreType.DMA,
        ],
    )(functools.partial(_deg_body, per_tile))


def _dinv(degp0, degp1):
    return lax.rsqrt(degp0[:, 0:1] + degp1[:, 0:1] + 1.0)


def _k0_body(degp_ref, x_ref, w_ref, g_ref):
    dinv = _dinv(degp_ref[0], degp_ref[1])
    g_ref[...] = dinv * jnp.dot(x_ref[...], w_ref[...],
                                preferred_element_type=jnp.float32)


def _k1_body(p_ref, g_ref, b_ref, degp_ref, w_ref, o_ref):
    dinv = _dinv(degp_ref[0], degp_ref[1])
    x = jnp.maximum(dinv * (p_ref[0] + p_ref[1] + g_ref[...]) + b_ref[...],
                    0.0)
    o_ref[...] = dinv * jnp.dot(x, w_ref[...],
                                preferred_element_type=jnp.float32)


def _k2_body(p_ref, g_ref, b_ref, degp_ref, w_ref, o_ref):
    dinv = _dinv(degp_ref[0], degp_ref[1])
    x = jnp.maximum(dinv * (p_ref[0] + p_ref[1] + g_ref[...]) + b_ref[...],
                    0.0)
    o_ref[...] = dinv * jnp.dot(x, w_ref[0],
                                preferred_element_type=jnp.float32)


def _k3_body(pp_ref, g_ref, b_ref, degp_ref, w_ref, o_ref):
    dinv = _dinv(degp_ref[0], degp_ref[1])
    x = jnp.maximum(dinv * (pp_ref[0] + g_ref[...]) + b_ref[0], 0.0)
    o_ref[...] = dinv * jnp.dot(x, w_ref[0],
                                preferred_element_type=jnp.float32)


def _k4_body(pa_ref, pb_ref, ga_ref, gb_ref, ba_ref, bb_ref, degp_ref,
             eps_ref, mu_ref, z_ref):
    dinv = _dinv(degp_ref[0], degp_ref[1])
    mu = jnp.maximum(dinv * (pa_ref[0] + ga_ref[...]) + ba_ref[...], 0.0)
    logvar = jnp.maximum(dinv * (pb_ref[0] + gb_ref[...]) + bb_ref[...], 0.0)
    mu_ref[...] = mu
    z_ref[...] = mu + eps_ref[...] * jnp.exp(0.5 * logvar)


def _dec_body(zr_ref, zc_ref, o_ref):
    o_ref[...] = lax.dot_general(
        zr_ref[...].astype(jnp.bfloat16), zc_ref[...].astype(jnp.bfloat16),
        (((1,), (1,)), ((), ())),
        preferred_element_type=jnp.float32)


def kernel(X, edge_index, W0, b0, W1, b1, W2, b2, W3, b3, W4, b4, W5, b5, W6,
           b6):
    f32 = jnp.float32
    src = edge_index[0]
    dst = edge_index[1]
    pad = E_PAD - E
    fill = jnp.arange(pad, dtype=jnp.int32)
    srcp = jnp.concatenate([src, (fill * 131) % N])
    dstp = jnp.concatenate([dst, N + fill % (NPAD - N)])
    src2 = srcp.reshape(NCHUNK, CH)
    dst2 = dstp.reshape(NCHUNK, CH)
    src_trunk = src2.reshape(NC, NCHUNK // NC, CH)
    dst_trunk = dst2.reshape(NC, NCHUNK // NC, CH)
    src_pair = jnp.stack([src2, src2 + N])
    dst_pair = jnp.stack([dst2, dst2])
    eps = jax.random.normal(jax.random.key(42), (N, D), dtype=f32)

    seg_trunk = _make_segsum(NCHUNK // NC)
    seg_pair = _make_segsum(NCHUNK)
    deg_k = _make_deg(NCHUNK // NC)

    degp = deg_k(dst_trunk)

    grid10 = N // BM
    spec_degp = pl.BlockSpec((NC, BM, DW), lambda i: (0, i, 0))
    spec_rows = pl.BlockSpec((BM, D), lambda i: (i, 0))
    spec_w = pl.BlockSpec((D, D), lambda i: (0, 0))
    spec_b = pl.BlockSpec((1, D), lambda i: (0, 0))
    spec_p = pl.BlockSpec((NC, BM, D), lambda i: (0, i, 0))

    b0r, b1r, b2r = b0.reshape(1, D), b1.reshape(1, D), b2.reshape(1, D)

    g = pl.pallas_call(
        _k0_body, grid=(grid10,),
        in_specs=[spec_degp, spec_rows, spec_w],
        out_specs=spec_rows,
        out_shape=jax.ShapeDtypeStruct((N, D), f32),
    )(degp, X, W0)

    for b_i, w_next in ((b0r, W1), (b1r, W2)):
        p = seg_trunk(g, src_trunk, dst_trunk)
        g = pl.pallas_call(
            _k1_body, grid=(grid10,),
            in_specs=[spec_p, spec_rows, spec_b, spec_degp, spec_w],
            out_specs=spec_rows,
            out_shape=jax.ShapeDtypeStruct((N, D), f32),
        )(p, g, b_i, degp, w_next)

    p2 = seg_trunk(g, src_trunk, dst_trunk)
    w35 = jnp.stack([W3, W5])
    g35 = pl.pallas_call(
        _k2_body, grid=(2 * grid10,),
        in_specs=[
            pl.BlockSpec((NC, BM, D), lambda i: (0, i % grid10, 0)),
            pl.BlockSpec((BM, D), lambda i: (i % grid10, 0)),
            spec_b,
            pl.BlockSpec((NC, BM, DW), lambda i: (0, i % grid10, 0)),
            pl.BlockSpec((1, D, D), lambda i: (i // grid10, 0, 0)),
        ],
        out_specs=pl.BlockSpec((BM, D), lambda i: (i, 0)),
        out_shape=jax.ShapeDtypeStruct((2 * N, D), f32),
    )(p2, g, b2r, degp, w35)

    pp = seg_pair(g35, src_pair, dst_pair)

    b35 = jnp.stack([b3.reshape(1, D), b5.reshape(1, D)])
    w46 = jnp.stack([W4, W6])
    g46 = pl.pallas_call(
        _k3_body, grid=(2 * grid10,),
        in_specs=[
            pl.BlockSpec((1, BM, D), lambda i: (i // grid10, i % grid10, 0)),
            pl.BlockSpec((BM, D), lambda i: (i, 0)),
            pl.BlockSpec((1, 1, D), lambda i: (i // grid10, 0, 0)),
            pl.BlockSpec((NC, BM, DW), lambda i: (0, i % grid10, 0)),
            pl.BlockSpec((1, D, D), lambda i: (i // grid10, 0, 0)),
        ],
        out_specs=pl.BlockSpec((BM, D), lambda i: (i, 0)),
        out_shape=jax.ShapeDtypeStruct((2 * N, D), f32),
    )(pp, g35, b35, degp, w46)

    pp2 = seg_pair(g46, src_pair, dst_pair)

    mu, z = pl.pallas_call(
        _k4_body, grid=(grid10,),
        in_specs=[
            pl.BlockSpec((1, BM, D), lambda i: (0, i, 0)),
            pl.BlockSpec((1, BM, D), lambda i: (1, i, 0)),
            pl.BlockSpec((BM, D), lambda i: (i, 0)),
            pl.BlockSpec((BM, D), lambda i: (grid10 + i, 0)),
            spec_b, spec_b, spec_degp, spec_rows,
        ],
        out_specs=[spec_rows, spec_rows],
        out_shape=[jax.ShapeDtypeStruct((N, D), f32),
                   jax.ShapeDtypeStruct((N, D), f32)],
    )(pp2, pp2, g46, g46, b4.reshape(1, D), b6.reshape(1, D), degp, eps)

    DM, DN = 2048, 2048
    adj = pl.pallas_call(
        _dec_body, grid=(pl.cdiv(N, DM), pl.cdiv(N, DN)),
        in_specs=[
            pl.BlockSpec((DM, D), lambda i, j: (i, 0)),
            pl.BlockSpec((DN, D), lambda i, j: (j, 0)),
        ],
        out_specs=pl.BlockSpec((DM, DN), lambda i, j: (i, j)),
        out_shape=jax.ShapeDtypeStruct((N, N), f32),
    )(z, z)

    return (adj, mu, mu)
